# Initial kernel scaffold; baseline (speedup 1.0000x reference)
import jax
import jax.numpy as jnp
from jax.experimental import pallas as pl


def kernel(x, edge_index, node_to_subgraph, subgraph_to_graph, W0, b0, W1, b1, W2, b2, W3, b3, Wc1, bc1, Wc2, bc2, Wl1, bl1, Wl2, bl2):
    raise NotImplementedError("write your pallas kernel here")


# speedup vs baseline: 11.8925x; 11.8925x over previous
import functools

import jax
import jax.numpy as jnp
from jax import lax
from jax.experimental import pallas as pl
from jax.experimental.pallas import tpu as pltpu
from jax.experimental.pallas import tpu_sc as plsc

N_NODES = 10000
N_EDGES = 320000
N_SUB = 1000
N_GRAPH = 100
D_FEAT = 128
HIDDEN = 32
K = 16
TOTAL_LATENT = 97

NPAD = 10240
PAD_NODE = N_NODES
NW = 32
CHUNK = 128
CPW = 80
E_PAD = NW * CPW * CHUNK
ROWS_PER_TILE = NPAD // 16


def _msgpass_body(src_hbm, dst_hbm, u_hbm, out_hbm, src_v, dst_v, rows_v, zbuf_v, acc_sh, sem):
    cid = lax.axis_index("c")
    sid = lax.axis_index("s")
    wid = cid * 16 + sid
    pltpu.sync_copy(src_hbm.at[pl.ds(wid * CPW, CPW)], src_v)
    pltpu.sync_copy(dst_hbm.at[pl.ds(wid * CPW, CPW)], dst_v)

    def zf(i, c):
        zbuf_v[i // 2, pl.ds((i % 2) * 16, 16)] = jnp.zeros((16,), jnp.float32)
        return c

    lax.fori_loop(0, 2 * CHUNK, zf, 0)
    for j in range(ROWS_PER_TILE // CHUNK):
        pltpu.sync_copy(zbuf_v, acc_sh.at[pl.ds(sid * ROWS_PER_TILE + j * CHUNK, CHUNK)])
    plsc.subcore_barrier()

    def body(c, carry):
        pltpu.async_copy(u_hbm.at[src_v.at[c]], rows_v, sem).wait()
        pltpu.sync_copy(rows_v, acc_sh.at[dst_v.at[c]], add=True)
        return carry

    lax.fori_loop(0, CPW, body, 0)
    plsc.subcore_barrier()
    pltpu.sync_copy(acc_sh.at[pl.ds(sid * ROWS_PER_TILE, ROWS_PER_TILE)],
                    out_hbm.at[cid, pl.ds(sid * ROWS_PER_TILE, ROWS_PER_TILE)])


@functools.lru_cache(maxsize=None)
def _msgpass_fn():
    return pl.kernel(
        _msgpass_body,
        out_type=jax.ShapeDtypeStruct((2, NPAD, HIDDEN), jnp.float32),
        mesh=plsc.VectorSubcoreMesh(core_axis_name="c", subcore_axis_name="s"),
        compiler_params=pltpu.CompilerParams(use_tc_tiling_on_sc=False),
        scratch_types=[
            pltpu.VMEM((CPW, CHUNK), jnp.int32),
            pltpu.VMEM((CPW, CHUNK), jnp.int32),
            pltpu.VMEM((CHUNK, HIDDEN), jnp.float32),
            pltpu.VMEM((CHUNK, HIDDEN), jnp.float32),
            pltpu.VMEM_SHARED((NPAD, HIDDEN), jnp.float32),
            pltpu.SemaphoreType.DMA,
        ],
    )


def _msgpass(src_p, dst_p, u):
    return _msgpass_fn()(src_p, dst_p, u)


def _tc_pre_body(x_ref, dp_ref, w0_ref, dinv_ref, u0_ref):
    dp = dp_ref[...]
    deg = 1.0 + dp[0, :, 0:1] + dp[1, :, 0:1]
    dinv = lax.rsqrt(deg)
    dinv_ref[...] = dinv
    u0_ref[...] = dinv * jnp.dot(x_ref[...], w0_ref[...], preferred_element_type=jnp.float32)


def _tc_pre(x_p, deg_parts, W0):
    return pl.pallas_call(
        _tc_pre_body,
        out_shape=(jax.ShapeDtypeStruct((NPAD, 1), jnp.float32),
                   jax.ShapeDtypeStruct((NPAD, HIDDEN), jnp.float32)),
    )(x_p, deg_parts, W0)


def _tc_layer_body(dp_ref, u_ref, dinv_ref, b_ref, wn_ref, h_ref, un_ref):
    dp = dp_ref[...]
    dinv = dinv_ref[...]
    h = jnp.tanh(dinv * (dp[0] + dp[1] + u_ref[...]) + b_ref[...][None, :])
    h_ref[...] = h
    un_ref[...] = dinv * jnp.dot(h, wn_ref[...], preferred_element_type=jnp.float32)


def _tc_layer(parts, u, dinv, b, Wn):
    return pl.pallas_call(
        _tc_layer_body,
        out_shape=(jax.ShapeDtypeStruct((NPAD, HIDDEN), jnp.float32),
                   jax.ShapeDtypeStruct((NPAD, HIDDEN), jnp.float32)),
    )(parts, u, dinv, b, Wn)


def _tc_last_body(dp_ref, u_ref, dinv_ref, b_ref, h_ref):
    dp = dp_ref[...]
    h_ref[...] = jnp.tanh(dinv_ref[...] * (dp[0] + dp[1] + u_ref[...]) + b_ref[...][None, :])


def _tc_last(parts, u, dinv, b):
    return pl.pallas_call(
        _tc_last_body,
        out_shape=jax.ShapeDtypeStruct((NPAD, HIDDEN), jnp.float32),
    )(parts, u, dinv, b)


def kernel(x, edge_index, node_to_subgraph, subgraph_to_graph,
           W0, b0, W1, b1, W2, b2, W3, b3,
           Wc1, bc1, Wc2, bc2, Wl1, bl1, Wl2, bl2):
    src, dst = edge_index[0], edge_index[1]
    pad_e = jnp.full((E_PAD - N_EDGES,), PAD_NODE, jnp.int32)
    src_p = jnp.concatenate([src, pad_e]).reshape(NW * CPW, CHUNK)
    dst_p = jnp.concatenate([dst, pad_e]).reshape(NW * CPW, CHUNK)
    x_p = jnp.pad(x, ((0, NPAD - N_NODES), (0, 0)))

    ones_u = jnp.ones((NPAD, HIDDEN), jnp.float32)
    deg_parts = _msgpass(src_p, dst_p, ones_u)
    dinv, u = _tc_pre(x_p, deg_parts, W0)

    W3p = jnp.pad(W3, ((0, 0), (0, HIDDEN - 1)))
    b3p = jnp.pad(b3, (0, HIDDEN - 1))
    hs = []
    for b, Wn in ((b0, W1), (b1, W2), (b2, W3p)):
        parts = _msgpass(src_p, dst_p, u)
        h, u = _tc_layer(parts, u, dinv, b, Wn)
        hs.append(h)
    parts = _msgpass(src_p, dst_p, u)
    h3 = _tc_last(parts, u, dinv, b3p)

    cs = jnp.concatenate([hs[0], hs[1], hs[2], h3[:, :1]], axis=1)[:N_NODES]

    counts = jnp.bincount(node_to_subgraph, length=N_SUB)
    starts = jnp.concatenate([jnp.zeros((1,), counts.dtype), jnp.cumsum(counts)[:-1]])
    order = jnp.lexsort((-cs[:, -1], node_to_subgraph))
    seg_sorted = node_to_subgraph[order]
    rank = jnp.arange(N_NODES) - starts[seg_sorted]
    dense = jnp.zeros((N_SUB, K, TOTAL_LATENT), jnp.float32)
    dense = dense.at[seg_sorted, rank].set(cs[order], mode='drop')

    z = jnp.einsum('skd,od->sok', dense, Wc1[:, 0, :]) + bc1[None, :, None]
    z = jax.nn.relu(z)
    z = z.reshape(N_SUB, 16, K // 2, 2).max(axis=-1)
    z = jax.lax.conv_general_dilated(z, Wc2, (1,), 'VALID',
                                     dimension_numbers=('NCH', 'OIH', 'NCH'))
    z = jax.nn.relu(z + bc2[None, :, None])
    z = z.reshape(N_SUB, -1)
    sums = jax.ops.segment_sum(z, subgraph_to_graph, num_segments=N_GRAPH)
    cnt = jax.ops.segment_sum(jnp.ones((N_SUB,), z.dtype), subgraph_to_graph, num_segments=N_GRAPH)
    g = sums / jnp.clip(cnt, 1.0)[:, None]
    g = jax.nn.relu(g @ Wl1 + bl1)
    out = g @ Wl2 + bl2
    return jax.nn.log_softmax(out, axis=-1)


# speedup vs baseline: 24.6533x; 2.0730x over previous
import functools

import jax
import jax.numpy as jnp
from jax import lax
from jax.experimental import pallas as pl
from jax.experimental.pallas import tpu as pltpu
from jax.experimental.pallas import tpu_sc as plsc

N_NODES = 10000
N_EDGES = 320000
N_SUB = 1000
N_GRAPH = 100
D_FEAT = 128
HIDDEN = 32
K = 16
TOTAL_LATENT = 97

NPAD = 10240
PAD_NODE = N_NODES
NW = 32
CHUNK = 128
CPW = 80
E_PAD = NW * CPW * CHUNK
ROWS_PER_TILE = NPAD // 16


def _msgpass_body(src_hbm, dst_hbm, u_hbm, out_hbm, src_v, dst_v, rows_v, zbuf_v, acc_sh, sem):
    cid = lax.axis_index("c")
    sid = lax.axis_index("s")
    wid = cid * 16 + sid
    pltpu.sync_copy(src_hbm.at[pl.ds(wid * CPW, CPW)], src_v)
    pltpu.sync_copy(dst_hbm.at[pl.ds(wid * CPW, CPW)], dst_v)

    def zf(i, c):
        zbuf_v[i // 2, pl.ds((i % 2) * 16, 16)] = jnp.zeros((16,), jnp.float32)
        return c

    lax.fori_loop(0, 2 * CHUNK, zf, 0)
    for j in range(ROWS_PER_TILE // CHUNK):
        pltpu.sync_copy(zbuf_v, acc_sh.at[pl.ds(sid * ROWS_PER_TILE + j * CHUNK, CHUNK)])
    plsc.subcore_barrier()

    pltpu.async_copy(u_hbm.at[src_v.at[0]], rows_v.at[0], sem)

    def body(c, carry):
        slot = lax.rem(c, 2)
        nxt = lax.rem(c + 1, 2)

        @pl.when(c + 1 < CPW)
        def _():
            pltpu.async_copy(u_hbm.at[src_v.at[c + 1]], rows_v.at[nxt], sem)

        pltpu.make_async_copy(u_hbm.at[src_v.at[c]], rows_v.at[slot], sem).wait()
        pltpu.sync_copy(rows_v.at[slot], acc_sh.at[dst_v.at[c]], add=True)
        return carry

    lax.fori_loop(0, CPW, body, 0)
    plsc.subcore_barrier()
    pltpu.sync_copy(acc_sh.at[pl.ds(sid * ROWS_PER_TILE, ROWS_PER_TILE)],
                    out_hbm.at[cid, pl.ds(sid * ROWS_PER_TILE, ROWS_PER_TILE)])


@functools.lru_cache(maxsize=None)
def _msgpass_fn():
    return pl.kernel(
        _msgpass_body,
        out_type=jax.ShapeDtypeStruct((2, NPAD, HIDDEN), jnp.float32),
        mesh=plsc.VectorSubcoreMesh(core_axis_name="c", subcore_axis_name="s"),
        compiler_params=pltpu.CompilerParams(use_tc_tiling_on_sc=False),
        scratch_types=[
            pltpu.VMEM((CPW, CHUNK), jnp.int32),
            pltpu.VMEM((CPW, CHUNK), jnp.int32),
            pltpu.VMEM((2, CHUNK, HIDDEN), jnp.float32),
            pltpu.VMEM((CHUNK, HIDDEN), jnp.float32),
            pltpu.VMEM_SHARED((NPAD, HIDDEN), jnp.float32),
            pltpu.SemaphoreType.DMA,
        ],
    )


def _msgpass(src_p, dst_p, u):
    return _msgpass_fn()(src_p, dst_p, u)


def _deg_body(dst_hbm, out_hbm, dst_v, ones_v, zbuf_v, acc_sh):
    cid = lax.axis_index("c")
    sid = lax.axis_index("s")
    wid = cid * 16 + sid
    pltpu.sync_copy(dst_hbm.at[pl.ds(wid * CPW, CPW)], dst_v)

    def zf(i, c):
        zbuf_v[pl.ds(i * 16, 16)] = jnp.zeros((16,), jnp.float32)
        ones_v[pl.ds(i * 16, 16)] = jnp.ones((16,), jnp.float32)
        return c

    lax.fori_loop(0, CHUNK // 16, zf, 0)
    for j in range(ROWS_PER_TILE // CHUNK):
        pltpu.sync_copy(zbuf_v, acc_sh.at[pl.ds(sid * ROWS_PER_TILE + j * CHUNK, CHUNK)])
    plsc.subcore_barrier()

    def body(c, carry):
        pltpu.sync_copy(ones_v, acc_sh.at[dst_v.at[c]], add=True)
        return carry

    lax.fori_loop(0, CPW, body, 0)
    plsc.subcore_barrier()
    pltpu.sync_copy(acc_sh.at[pl.ds(sid * ROWS_PER_TILE, ROWS_PER_TILE)],
                    out_hbm.at[cid, pl.ds(sid * ROWS_PER_TILE, ROWS_PER_TILE)])


@functools.lru_cache(maxsize=None)
def _deg_fn():
    return pl.kernel(
        _deg_body,
        out_type=jax.ShapeDtypeStruct((2, NPAD), jnp.float32),
        mesh=plsc.VectorSubcoreMesh(core_axis_name="c", subcore_axis_name="s"),
        compiler_params=pltpu.CompilerParams(use_tc_tiling_on_sc=False),
        scratch_types=[
            pltpu.VMEM((CPW, CHUNK), jnp.int32),
            pltpu.VMEM((CHUNK,), jnp.float32),
            pltpu.VMEM((CHUNK,), jnp.float32),
            pltpu.VMEM_SHARED((NPAD,), jnp.float32),
        ],
    )


def _deg(dst_p):
    return _deg_fn()(dst_p)


def _tc_pre_body(x_ref, dp_ref, w0_ref, dinv_ref, u0_ref):
    dp = dp_ref[...]
    deg = 1.0 + (dp[0] + dp[1])[:, None]
    dinv = lax.rsqrt(deg)
    dinv_ref[...] = dinv
    u0_ref[...] = dinv * jnp.dot(x_ref[...], w0_ref[...], preferred_element_type=jnp.float32)


def _tc_pre(x_p, deg_parts, W0):
    return pl.pallas_call(
        _tc_pre_body,
        out_shape=(jax.ShapeDtypeStruct((NPAD, 1), jnp.float32),
                   jax.ShapeDtypeStruct((NPAD, HIDDEN), jnp.float32)),
    )(x_p, deg_parts, W0)


def _tc_layer_body(dp_ref, u_ref, dinv_ref, b_ref, wn_ref, h_ref, un_ref):
    dp = dp_ref[...]
    dinv = dinv_ref[...]
    h = jnp.tanh(dinv * (dp[0] + dp[1] + u_ref[...]) + b_ref[...][None, :])
    h_ref[...] = h
    un_ref[...] = dinv * jnp.dot(h, wn_ref[...], preferred_element_type=jnp.float32)


def _tc_layer(parts, u, dinv, b, Wn):
    return pl.pallas_call(
        _tc_layer_body,
        out_shape=(jax.ShapeDtypeStruct((NPAD, HIDDEN), jnp.float32),
                   jax.ShapeDtypeStruct((NPAD, HIDDEN), jnp.float32)),
    )(parts, u, dinv, b, Wn)


def _tc_last_body(dp_ref, u_ref, dinv_ref, b_ref, h_ref):
    dp = dp_ref[...]
    h_ref[...] = jnp.tanh(dinv_ref[...] * (dp[0] + dp[1] + u_ref[...]) + b_ref[...][None, :])


def _tc_last(parts, u, dinv, b):
    return pl.pallas_call(
        _tc_last_body,
        out_shape=jax.ShapeDtypeStruct((NPAD, HIDDEN), jnp.float32),
    )(parts, u, dinv, b)


def kernel(x, edge_index, node_to_subgraph, subgraph_to_graph,
           W0, b0, W1, b1, W2, b2, W3, b3,
           Wc1, bc1, Wc2, bc2, Wl1, bl1, Wl2, bl2):
    src, dst = edge_index[0], edge_index[1]
    pad_e = PAD_NODE + jnp.arange(E_PAD - N_EDGES, dtype=jnp.int32) % (NPAD - N_NODES)
    src_p = jnp.concatenate([src, pad_e]).reshape(NW * CPW, CHUNK)
    dst_p = jnp.concatenate([dst, pad_e]).reshape(NW * CPW, CHUNK)
    x_p = jnp.pad(x, ((0, NPAD - N_NODES), (0, 0)))

    deg_parts = _deg(dst_p)
    dinv, u = _tc_pre(x_p, deg_parts, W0)

    W3p = jnp.pad(W3, ((0, 0), (0, HIDDEN - 1)))
    b3p = jnp.pad(b3, (0, HIDDEN - 1))
    hs = []
    for b, Wn in ((b0, W1), (b1, W2), (b2, W3p)):
        parts = _msgpass(src_p, dst_p, u)
        h, u = _tc_layer(parts, u, dinv, b, Wn)
        hs.append(h)
    parts = _msgpass(src_p, dst_p, u)
    h3 = _tc_last(parts, u, dinv, b3p)

    cs = jnp.concatenate([hs[0], hs[1], hs[2], h3[:, :1]], axis=1)[:N_NODES]

    counts = jnp.bincount(node_to_subgraph, length=N_SUB)
    starts = jnp.concatenate([jnp.zeros((1,), counts.dtype), jnp.cumsum(counts)[:-1]])
    order = jnp.lexsort((-cs[:, -1], node_to_subgraph))
    seg_sorted = node_to_subgraph[order]
    rank = jnp.arange(N_NODES) - starts[seg_sorted]
    dense = jnp.zeros((N_SUB, K, TOTAL_LATENT), jnp.float32)
    dense = dense.at[seg_sorted, rank].set(cs[order], mode='drop')

    z = jnp.einsum('skd,od->sok', dense, Wc1[:, 0, :]) + bc1[None, :, None]
    z = jax.nn.relu(z)
    z = z.reshape(N_SUB, 16, K // 2, 2).max(axis=-1)
    z = jax.lax.conv_general_dilated(z, Wc2, (1,), 'VALID',
                                     dimension_numbers=('NCH', 'OIH', 'NCH'))
    z = jax.nn.relu(z + bc2[None, :, None])
    z = z.reshape(N_SUB, -1)
    sums = jax.ops.segment_sum(z, subgraph_to_graph, num_segments=N_GRAPH)
    cnt = jax.ops.segment_sum(jnp.ones((N_SUB,), z.dtype), subgraph_to_graph, num_segments=N_GRAPH)
    g = sums / jnp.clip(cnt, 1.0)[:, None]
    g = jax.nn.relu(g @ Wl1 + bl1)
    out = g @ Wl2 + bl2
    return jax.nn.log_softmax(out, axis=-1)


# speedup vs baseline: 31.1561x; 1.2638x over previous
import functools

import jax
import jax.numpy as jnp
from jax import lax
from jax.experimental import pallas as pl
from jax.experimental.pallas import tpu as pltpu
from jax.experimental.pallas import tpu_sc as plsc

N_NODES = 10000
N_EDGES = 320000
N_SUB = 1000
N_GRAPH = 100
D_FEAT = 128
HIDDEN = 32
K = 16
TOTAL_LATENT = 97
NUM_CLASSES = 10

NPAD = 10240
PAD_NODE = N_NODES
NW = 32
CHUNK = 128
CPW = 80
E_PAD = NW * CPW * CHUNK
ROWS_PER_TILE = NPAD // 16


def _msgpass_body(src_hbm, dst_hbm, u_hbm, out_hbm, src_v, dst_v, rows_v, zbuf_v, acc_sh, sem):
    cid = lax.axis_index("c")
    sid = lax.axis_index("s")
    wid = cid * 16 + sid
    pltpu.sync_copy(src_hbm.at[pl.ds(wid * CPW, CPW)], src_v)
    pltpu.sync_copy(dst_hbm.at[pl.ds(wid * CPW, CPW)], dst_v)

    def zf(i, c):
        zbuf_v[i // 2, pl.ds((i % 2) * 16, 16)] = jnp.zeros((16,), jnp.float32)
        return c

    lax.fori_loop(0, 2 * CHUNK, zf, 0)
    for j in range(ROWS_PER_TILE // CHUNK):
        pltpu.sync_copy(zbuf_v, acc_sh.at[pl.ds(sid * ROWS_PER_TILE + j * CHUNK, CHUNK)])
    plsc.subcore_barrier()

    pltpu.async_copy(u_hbm.at[src_v.at[0]], rows_v.at[0], sem)

    def body(c, carry):
        slot = lax.rem(c, 2)
        nxt = lax.rem(c + 1, 2)

        @pl.when(c + 1 < CPW)
        def _():
            pltpu.async_copy(u_hbm.at[src_v.at[c + 1]], rows_v.at[nxt], sem)

        pltpu.make_async_copy(u_hbm.at[src_v.at[c]], rows_v.at[slot], sem).wait()
        pltpu.sync_copy(rows_v.at[slot], acc_sh.at[dst_v.at[c]], add=True)
        return carry

    lax.fori_loop(0, CPW, body, 0)
    plsc.subcore_barrier()
    pltpu.sync_copy(acc_sh.at[pl.ds(sid * ROWS_PER_TILE, ROWS_PER_TILE)],
                    out_hbm.at[cid, pl.ds(sid * ROWS_PER_TILE, ROWS_PER_TILE)])


@functools.lru_cache(maxsize=None)
def _msgpass_fn():
    return pl.kernel(
        _msgpass_body,
        out_type=jax.ShapeDtypeStruct((2, NPAD, HIDDEN), jnp.float32),
        mesh=plsc.VectorSubcoreMesh(core_axis_name="c", subcore_axis_name="s"),
        compiler_params=pltpu.CompilerParams(use_tc_tiling_on_sc=False),
        scratch_types=[
            pltpu.VMEM((CPW, CHUNK), jnp.int32),
            pltpu.VMEM((CPW, CHUNK), jnp.int32),
            pltpu.VMEM((2, CHUNK, HIDDEN), jnp.float32),
            pltpu.VMEM((CHUNK, HIDDEN), jnp.float32),
            pltpu.VMEM_SHARED((NPAD, HIDDEN), jnp.float32),
            pltpu.SemaphoreType.DMA,
        ],
    )


def _msgpass(src_p, dst_p, u):
    return _msgpass_fn()(src_p, dst_p, u)


def _deg_body(dst_hbm, out_hbm, dst_v, ones_v, zbuf_v, acc_sh):
    cid = lax.axis_index("c")
    sid = lax.axis_index("s")
    wid = cid * 16 + sid
    pltpu.sync_copy(dst_hbm.at[pl.ds(wid * CPW, CPW)], dst_v)

    def zf(i, c):
        zbuf_v[pl.ds(i * 16, 16)] = jnp.zeros((16,), jnp.float32)
        ones_v[pl.ds(i * 16, 16)] = jnp.ones((16,), jnp.float32)
        return c

    lax.fori_loop(0, CHUNK // 16, zf, 0)
    for j in range(ROWS_PER_TILE // CHUNK):
        pltpu.sync_copy(zbuf_v, acc_sh.at[pl.ds(sid * ROWS_PER_TILE + j * CHUNK, CHUNK)])
    plsc.subcore_barrier()

    def body(c, carry):
        pltpu.sync_copy(ones_v, acc_sh.at[dst_v.at[c]], add=True)
        return carry

    lax.fori_loop(0, CPW, body, 0)
    plsc.subcore_barrier()
    pltpu.sync_copy(acc_sh.at[pl.ds(sid * ROWS_PER_TILE, ROWS_PER_TILE)],
                    out_hbm.at[cid, pl.ds(sid * ROWS_PER_TILE, ROWS_PER_TILE)])


@functools.lru_cache(maxsize=None)
def _deg_fn():
    return pl.kernel(
        _deg_body,
        out_type=jax.ShapeDtypeStruct((2, NPAD), jnp.float32),
        mesh=plsc.VectorSubcoreMesh(core_axis_name="c", subcore_axis_name="s"),
        compiler_params=pltpu.CompilerParams(use_tc_tiling_on_sc=False),
        scratch_types=[
            pltpu.VMEM((CPW, CHUNK), jnp.int32),
            pltpu.VMEM((CHUNK,), jnp.float32),
            pltpu.VMEM((CHUNK,), jnp.float32),
            pltpu.VMEM_SHARED((NPAD,), jnp.float32),
        ],
    )


def _deg(dst_p):
    return _deg_fn()(dst_p)


NSTART = 1040


def _tc_pre_body(x_ref, dp_ref, w0_ref, n2s_ref, dinv_ref, u0_ref, starts_ref):
    dp = dp_ref[...]
    deg = 1.0 + (dp[0] + dp[1])[:, None]
    dinv = lax.rsqrt(deg)
    dinv_ref[...] = dinv
    u0_ref[...] = dinv * jnp.dot(x_ref[...], w0_ref[...], preferred_element_type=jnp.float32)
    s_iota = lax.broadcasted_iota(jnp.int32, (NSTART, 1), 0)
    acc = jnp.zeros((NSTART,), jnp.float32)
    for c in range(NPAD // 1024):
        chunk = n2s_ref[pl.ds(c * 1024, 1024)]
        acc = acc + jnp.sum((chunk[None, :] < s_iota).astype(jnp.float32), axis=1)
    starts_ref[...] = acc.astype(jnp.int32)


def _tc_pre(x_p, deg_parts, W0, n2s_pad):
    return pl.pallas_call(
        _tc_pre_body,
        out_shape=(jax.ShapeDtypeStruct((NPAD, 1), jnp.float32),
                   jax.ShapeDtypeStruct((NPAD, HIDDEN), jnp.float32),
                   jax.ShapeDtypeStruct((NSTART,), jnp.int32)),
    )(x_p, deg_parts, W0, n2s_pad)


def _tc_layer_body(dp_ref, u_ref, dinv_ref, b_ref, wn_ref, h_ref, un_ref):
    dp = dp_ref[...]
    dinv = dinv_ref[...]
    h = jnp.tanh(dinv * (dp[0] + dp[1] + u_ref[...]) + b_ref[...][None, :])
    h_ref[...] = h
    un_ref[...] = dinv * jnp.dot(h, wn_ref[...], preferred_element_type=jnp.float32)


def _tc_layer(parts, u, dinv, b, Wn):
    return pl.pallas_call(
        _tc_layer_body,
        out_shape=(jax.ShapeDtypeStruct((NPAD, HIDDEN), jnp.float32),
                   jax.ShapeDtypeStruct((NPAD, HIDDEN), jnp.float32)),
    )(parts, u, dinv, b, Wn)


D2 = 104
VCOL_LEN = NPAD + 512


def _tc_cs_body(dp_ref, u_ref, dinv_ref, b_ref, h0_ref, h1_ref, h2_ref,
                cs_ref, vcol_ref):
    dp = dp_ref[...]
    h3 = jnp.tanh(dinv_ref[...] * (dp[0] + dp[1] + u_ref[...]) + b_ref[...][None, :])
    row = lax.broadcasted_iota(jnp.int32, (NPAD, 1), 0)
    valid = (row < N_NODES).astype(jnp.float32)
    cs = jnp.concatenate(
        [h0_ref[...], h1_ref[...], h2_ref[...], h3[:, 0:1],
         jnp.zeros((NPAD, D2 - TOTAL_LATENT), jnp.float32)], axis=1)
    cs = cs * valid
    cs_ref[...] = cs
    vcol_ref[...] = jnp.concatenate([cs[:, TOTAL_LATENT - 1],
                                     jnp.zeros((VCOL_LEN - NPAD,), jnp.float32)])


def _tc_cs(parts, u, dinv, b, h0, h1, h2):
    return pl.pallas_call(
        _tc_cs_body,
        out_shape=(jax.ShapeDtypeStruct((NPAD, D2), jnp.float32),
                   jax.ShapeDtypeStruct((VCOL_LEN,), jnp.float32)),
    )(parts, u, dinv, b, h0, h1, h2)


SEGS_PER_W = 32
NSUB_PAD = 1024


def _sortpool_body(cs_hbm, vcol_hbm, starts_hbm, out_hbm,
                   starts_v, vbuf_v, keys_v, ids_v, rows_v, sem):
    cid = lax.axis_index("c")
    sid = lax.axis_index("s")
    w = cid * 16 + sid
    pltpu.sync_copy(starts_hbm.at[pl.ds(w * SEGS_PER_W, 40)], starts_v.at[pl.ds(0, 40)])
    s0 = starts_v[pl.ds(0, 16)][0]
    endw = starts_v[pl.ds(SEGS_PER_W, 16)][0]
    base8 = (s0 // 8) * 8
    nld = (endw - base8 + 511) // 512

    def ld(j, c):
        pltpu.sync_copy(vcol_hbm.at[pl.ds(base8 + j * 512, 512)],
                        vbuf_v.at[pl.ds(j * 512, 512)])
        return c

    lax.fori_loop(0, nld, ld, 0)
    lane = lax.iota(jnp.int32, 16)

    def seg_body(s, carry):
        sg = w * SEGS_PER_W + s
        stv = starts_v[pl.ds(s, 16)]
        st = stv[0]
        en = stv[1]
        zr = N_NODES + lax.rem(sg, NPAD - N_NODES)
        keys_v[...] = jnp.full((16,), 3.0, jnp.float32)
        ids_v[...] = jnp.broadcast_to(zr, (16,))

        def node_body(i, c):
            v = -vbuf_v[pl.ds(i - base8, 16)][0]
            kk = keys_v[...]
            ii = ids_v[...]
            cond = jnp.logical_and(lane == 15, v < kk[15])
            nk = jnp.where(cond, v, kk)
            ni = jnp.where(cond, i, ii)
            snk, sni = lax.sort([nk, ni], dimension=0, num_keys=1)
            keys_v[...] = snk
            ids_v[...] = sni
            return c

        lax.fori_loop(st, en, node_body, 0)
        pltpu.async_copy(cs_hbm.at[ids_v], rows_v, sem).wait()
        pltpu.sync_copy(rows_v, out_hbm.at[pl.ds(sg * 16, 16)])
        return carry

    lax.fori_loop(0, SEGS_PER_W, seg_body, 0)


@functools.lru_cache(maxsize=None)
def _sortpool_fn():
    return pl.kernel(
        _sortpool_body,
        out_type=jax.ShapeDtypeStruct((NSUB_PAD * 16, D2), jnp.float32),
        mesh=plsc.VectorSubcoreMesh(core_axis_name="c", subcore_axis_name="s"),
        compiler_params=pltpu.CompilerParams(use_tc_tiling_on_sc=False,
                                             needs_layout_passes=False),
        scratch_types=[
            pltpu.VMEM((64,), jnp.int32),
            pltpu.VMEM((VCOL_LEN,), jnp.float32),
            pltpu.VMEM((16,), jnp.float32),
            pltpu.VMEM((16,), jnp.int32),
            pltpu.VMEM((16, D2), jnp.float32),
            pltpu.SemaphoreType.DMA,
        ],
    )


def _sortpool(cs, vcol, starts):
    return _sortpool_fn()(cs, vcol, starts)


def _tc_head_body(d_ref, w1_ref, bc1_ref, w2_ref, bc2_ref, wl1_ref, bl1_ref,
                  wl2_ref, bl2_ref, s2g_ref, out_ref):
    z = jnp.dot(d_ref[...], w1_ref[...], preferred_element_type=jnp.float32)
    z = jax.nn.relu(z + bc1_ref[...][None, :])
    zm = z.reshape(NSUB_PAD * 8, 2, 16).max(axis=1)
    zm = zm.reshape(NSUB_PAD, 8, 16)
    zc = jnp.concatenate([zm[:, t:t + 4, :] for t in range(5)], axis=2)
    z2 = jnp.dot(zc.reshape(NSUB_PAD * 4, 80), w2_ref[...],
                 preferred_element_type=jnp.float32)
    z2 = jax.nn.relu(z2 + bc2_ref[...][None, :])
    z2v = z2.reshape(NSUB_PAD, 4, 32)
    g_iota = lax.broadcasted_iota(jnp.int32, (N_GRAPH, NSUB_PAD), 0)
    m = (s2g_ref[...][None, :] == g_iota).astype(jnp.float32)
    blocks = [jnp.dot(m, z2v[:, p, :], preferred_element_type=jnp.float32)
              for p in range(4)]
    sums = jnp.concatenate(blocks, axis=1)
    cnt = jnp.sum(m, axis=1, keepdims=True)
    g = sums / jnp.maximum(cnt, 1.0)
    g = jax.nn.relu(jnp.dot(g, wl1_ref[...], preferred_element_type=jnp.float32)
                    + bl1_ref[...][None, :])
    o = jnp.dot(g, wl2_ref[...], preferred_element_type=jnp.float32) + bl2_ref[...][None, :]
    mx = jnp.max(o, axis=-1, keepdims=True)
    lse = mx + jnp.log(jnp.sum(jnp.exp(o - mx), axis=-1, keepdims=True))
    out_ref[...] = o - lse


def _tc_head(dense, W1m, bc1, W2m, bc2, Wl1p, bl1, Wl2, bl2, s2g_pad):
    return pl.pallas_call(
        _tc_head_body,
        out_shape=jax.ShapeDtypeStruct((N_GRAPH, NUM_CLASSES), jnp.float32),
    )(dense, W1m, bc1, W2m, bc2, Wl1p, bl1, Wl2, bl2, s2g_pad)


def kernel(x, edge_index, node_to_subgraph, subgraph_to_graph,
           W0, b0, W1, b1, W2, b2, W3, b3,
           Wc1, bc1, Wc2, bc2, Wl1, bl1, Wl2, bl2):
    src, dst = edge_index[0], edge_index[1]
    pad_e = PAD_NODE + jnp.arange(E_PAD - N_EDGES, dtype=jnp.int32) % (NPAD - N_NODES)
    src_p = jnp.concatenate([src, pad_e]).reshape(NW * CPW, CHUNK)
    dst_p = jnp.concatenate([dst, pad_e]).reshape(NW * CPW, CHUNK)
    x_p = jnp.pad(x, ((0, NPAD - N_NODES), (0, 0)))

    n2s_pad = jnp.concatenate([node_to_subgraph,
                               jnp.full((NPAD - N_NODES,), 2000, jnp.int32)])
    deg_parts = _deg(dst_p)
    dinv, u, starts = _tc_pre(x_p, deg_parts, W0, n2s_pad)

    W3p = jnp.pad(W3, ((0, 0), (0, HIDDEN - 1)))
    b3p = jnp.pad(b3, (0, HIDDEN - 1))
    hs = []
    for b, Wn in ((b0, W1), (b1, W2), (b2, W3p)):
        parts = _msgpass(src_p, dst_p, u)
        h, u = _tc_layer(parts, u, dinv, b, Wn)
        hs.append(h)
    parts = _msgpass(src_p, dst_p, u)
    cs, vcol = _tc_cs(parts, u, dinv, b3p, hs[0], hs[1], hs[2])

    dense = _sortpool(cs, vcol, starts)

    W1m = jnp.pad(Wc1[:, 0, :].T, ((0, D2 - TOTAL_LATENT), (0, 0)))
    W2m = Wc2.transpose(2, 1, 0).reshape(80, 32)
    f = jnp.arange(128)
    perm = (f % 32) * 4 + f // 32
    Wl1p = Wl1[perm, :]
    s2g_pad = jnp.concatenate([subgraph_to_graph,
                               jnp.full((NSUB_PAD - N_SUB,), 1000, jnp.int32)])
    return _tc_head(dense, W1m, bc1, W2m, bc2, Wl1p, bl1, Wl2, bl2, s2g_pad)


# speedup vs baseline: 33.8023x; 1.0849x over previous
import functools

import jax
import jax.numpy as jnp
from jax import lax
from jax.experimental import pallas as pl
from jax.experimental.pallas import tpu as pltpu
from jax.experimental.pallas import tpu_sc as plsc

N_NODES = 10000
N_EDGES = 320000
N_SUB = 1000
N_GRAPH = 100
D_FEAT = 128
HIDDEN = 32
K = 16
TOTAL_LATENT = 97
NUM_CLASSES = 10

NPAD = 10240
NBUF = 6
PAD_NODE = N_NODES
NW = 32
CHUNK = 128
CPW = 80
E_PAD = NW * CPW * CHUNK
ROWS_PER_TILE = NPAD // 16


def _msgpass_body(src_hbm, dst_hbm, u_hbm, out_hbm, src_v, dst_v, rows_v, zbuf_v, acc_sh, sem_g, sem_s):
    cid = lax.axis_index("c")
    sid = lax.axis_index("s")
    wid = cid * 16 + sid
    pltpu.sync_copy(src_hbm.at[pl.ds(wid * CPW, CPW)], src_v)
    pltpu.sync_copy(dst_hbm.at[pl.ds(wid * CPW, CPW)], dst_v)

    def zf(i, c):
        zbuf_v[i // 2, pl.ds((i % 2) * 16, 16)] = jnp.zeros((16,), jnp.float32)
        return c

    lax.fori_loop(0, 2 * CHUNK, zf, 0)
    for j in range(ROWS_PER_TILE // CHUNK):
        pltpu.sync_copy(zbuf_v, acc_sh.at[pl.ds(sid * ROWS_PER_TILE + j * CHUNK, CHUNK)])
    plsc.subcore_barrier()

    pltpu.async_copy(u_hbm.at[src_v.at[0]], rows_v.at[0], sem_g)
    pltpu.async_copy(u_hbm.at[src_v.at[1]], rows_v.at[1], sem_g)

    def body(c, carry):
        slot = lax.rem(c, NBUF)
        pltpu.make_async_copy(u_hbm.at[src_v.at[c]], rows_v.at[slot], sem_g).wait()
        pltpu.async_copy(rows_v.at[slot], acc_sh.at[dst_v.at[c]], sem_s, add=True)

        @pl.when(c + 2 < CPW)
        def _():
            @pl.when(c >= NBUF - 2)
            def _():
                old = c - (NBUF - 2)
                pltpu.make_async_copy(rows_v.at[lax.rem(old, NBUF)],
                                      acc_sh.at[dst_v.at[old]], sem_s).wait()
            pltpu.async_copy(u_hbm.at[src_v.at[c + 2]],
                             rows_v.at[lax.rem(c + 2, NBUF)], sem_g)
        return carry

    lax.fori_loop(0, CPW, body, 0)

    def drain(j, carry):
        pltpu.make_async_copy(rows_v.at[lax.rem(j, NBUF)],
                              acc_sh.at[dst_v.at[j]], sem_s).wait()
        return carry

    lax.fori_loop(CPW - NBUF, CPW, drain, 0)
    plsc.subcore_barrier()
    pltpu.sync_copy(acc_sh.at[pl.ds(sid * ROWS_PER_TILE, ROWS_PER_TILE)],
                    out_hbm.at[cid, pl.ds(sid * ROWS_PER_TILE, ROWS_PER_TILE)])


@functools.lru_cache(maxsize=None)
def _msgpass_fn():
    return pl.kernel(
        _msgpass_body,
        out_type=jax.ShapeDtypeStruct((2, NPAD, HIDDEN), jnp.float32),
        mesh=plsc.VectorSubcoreMesh(core_axis_name="c", subcore_axis_name="s"),
        compiler_params=pltpu.CompilerParams(use_tc_tiling_on_sc=False),
        scratch_types=[
            pltpu.VMEM((CPW, CHUNK), jnp.int32),
            pltpu.VMEM((CPW, CHUNK), jnp.int32),
            pltpu.VMEM((NBUF, CHUNK, HIDDEN), jnp.float32),
            pltpu.VMEM((CHUNK, HIDDEN), jnp.float32),
            pltpu.VMEM_SHARED((NPAD, HIDDEN), jnp.float32),
            pltpu.SemaphoreType.DMA,
            pltpu.SemaphoreType.DMA,
        ],
    )


def _msgpass(src_p, dst_p, u):
    return _msgpass_fn()(src_p, dst_p, u)


def _deg_body(dst_hbm, out_hbm, dst_v, ones_v, zbuf_v, acc_sh):
    cid = lax.axis_index("c")
    sid = lax.axis_index("s")
    wid = cid * 16 + sid
    pltpu.sync_copy(dst_hbm.at[pl.ds(wid * CPW, CPW)], dst_v)

    def zf(i, c):
        zbuf_v[pl.ds(i * 16, 16)] = jnp.zeros((16,), jnp.float32)
        ones_v[pl.ds(i * 16, 16)] = jnp.ones((16,), jnp.float32)
        return c

    lax.fori_loop(0, CHUNK // 16, zf, 0)
    for j in range(ROWS_PER_TILE // CHUNK):
        pltpu.sync_copy(zbuf_v, acc_sh.at[pl.ds(sid * ROWS_PER_TILE + j * CHUNK, CHUNK)])
    plsc.subcore_barrier()

    def body(c, carry):
        pltpu.sync_copy(ones_v, acc_sh.at[dst_v.at[c]], add=True)
        return carry

    lax.fori_loop(0, CPW, body, 0)
    plsc.subcore_barrier()
    pltpu.sync_copy(acc_sh.at[pl.ds(sid * ROWS_PER_TILE, ROWS_PER_TILE)],
                    out_hbm.at[cid, pl.ds(sid * ROWS_PER_TILE, ROWS_PER_TILE)])


@functools.lru_cache(maxsize=None)
def _deg_fn():
    return pl.kernel(
        _deg_body,
        out_type=jax.ShapeDtypeStruct((2, NPAD), jnp.float32),
        mesh=plsc.VectorSubcoreMesh(core_axis_name="c", subcore_axis_name="s"),
        compiler_params=pltpu.CompilerParams(use_tc_tiling_on_sc=False),
        scratch_types=[
            pltpu.VMEM((CPW, CHUNK), jnp.int32),
            pltpu.VMEM((CHUNK,), jnp.float32),
            pltpu.VMEM((CHUNK,), jnp.float32),
            pltpu.VMEM_SHARED((NPAD,), jnp.float32),
        ],
    )


def _deg(dst_p):
    return _deg_fn()(dst_p)


NSTART = 1040


def _tc_pre_body(x_ref, dp_ref, w0_ref, n2s_ref, dinv_ref, u0_ref, starts_ref):
    dp = dp_ref[...]
    deg = 1.0 + (dp[0] + dp[1])[:, None]
    dinv = lax.rsqrt(deg)
    dinv_ref[...] = dinv
    u0_ref[...] = dinv * jnp.dot(x_ref[...], w0_ref[...], preferred_element_type=jnp.float32)
    s_iota = lax.broadcasted_iota(jnp.int32, (NSTART, 1), 0)
    acc = jnp.zeros((NSTART,), jnp.float32)
    for c in range(NPAD // 1024):
        chunk = n2s_ref[pl.ds(c * 1024, 1024)]
        acc = acc + jnp.sum((chunk[None, :] < s_iota).astype(jnp.float32), axis=1)
    starts_ref[...] = acc.astype(jnp.int32)


def _tc_pre(x_p, deg_parts, W0, n2s_pad):
    return pl.pallas_call(
        _tc_pre_body,
        out_shape=(jax.ShapeDtypeStruct((NPAD, 1), jnp.float32),
                   jax.ShapeDtypeStruct((NPAD, HIDDEN), jnp.float32),
                   jax.ShapeDtypeStruct((NSTART,), jnp.int32)),
    )(x_p, deg_parts, W0, n2s_pad)


def _tc_layer_body(dp_ref, u_ref, dinv_ref, b_ref, wn_ref, h_ref, un_ref):
    dp = dp_ref[...]
    dinv = dinv_ref[...]
    h = jnp.tanh(dinv * (dp[0] + dp[1] + u_ref[...]) + b_ref[...][None, :])
    h_ref[...] = h
    un_ref[...] = dinv * jnp.dot(h, wn_ref[...], preferred_element_type=jnp.float32)


def _tc_layer(parts, u, dinv, b, Wn):
    return pl.pallas_call(
        _tc_layer_body,
        out_shape=(jax.ShapeDtypeStruct((NPAD, HIDDEN), jnp.float32),
                   jax.ShapeDtypeStruct((NPAD, HIDDEN), jnp.float32)),
    )(parts, u, dinv, b, Wn)


D2 = 104
VCOL_LEN = NPAD + 512


def _tc_cs_body(dp_ref, u_ref, dinv_ref, b_ref, h0_ref, h1_ref, h2_ref,
                cs_ref, vcol_ref):
    dp = dp_ref[...]
    h3 = jnp.tanh(dinv_ref[...] * (dp[0] + dp[1] + u_ref[...]) + b_ref[...][None, :])
    row = lax.broadcasted_iota(jnp.int32, (NPAD, 1), 0)
    valid = (row < N_NODES).astype(jnp.float32)
    cs = jnp.concatenate(
        [h0_ref[...], h1_ref[...], h2_ref[...], h3[:, 0:1],
         jnp.zeros((NPAD, D2 - TOTAL_LATENT), jnp.float32)], axis=1)
    cs = cs * valid
    cs_ref[...] = cs
    vcol_ref[...] = jnp.concatenate([cs[:, TOTAL_LATENT - 1],
                                     jnp.zeros((VCOL_LEN - NPAD,), jnp.float32)])


def _tc_cs(parts, u, dinv, b, h0, h1, h2):
    return pl.pallas_call(
        _tc_cs_body,
        out_shape=(jax.ShapeDtypeStruct((NPAD, D2), jnp.float32),
                   jax.ShapeDtypeStruct((VCOL_LEN,), jnp.float32)),
    )(parts, u, dinv, b, h0, h1, h2)


SEGS_PER_W = 32
NSUB_PAD = 1024


def _sortpool_body(cs_hbm, vcol_hbm, starts_hbm, out_hbm,
                   starts_v, vbuf_v, ids_v, rows_v, sem_g, sem_o):
    cid = lax.axis_index("c")
    sid = lax.axis_index("s")
    w = cid * 16 + sid
    pltpu.sync_copy(starts_hbm.at[pl.ds(w * SEGS_PER_W, 40)], starts_v.at[pl.ds(0, 40)])
    s0 = starts_v[pl.ds(0, 16)][0]
    endw = starts_v[pl.ds(SEGS_PER_W, 16)][0]
    base8 = (s0 // 8) * 8
    nld = (endw - base8 + 511) // 512

    def ld(j, c):
        pltpu.sync_copy(vcol_hbm.at[pl.ds(base8 + j * 512, 512)],
                        vbuf_v.at[pl.ds(j * 512, 512)])
        return c

    lax.fori_loop(0, nld, ld, 0)
    lane = lax.iota(jnp.int32, 16)

    def seg_body(s, carry):
        sg = w * SEGS_PER_W + s
        slot = lax.rem(s, 2)
        stv = starts_v[pl.ds(s, 16)]
        st = stv[0]
        en = stv[1]
        zr = N_NODES + lax.rem(sg, NPAD - N_NODES)
        keys0 = jnp.full((16,), 3.0, jnp.float32)
        ids0 = jnp.broadcast_to(zr, (16,))

        def node_body(i, kki):
            kk, ii = kki
            v = -vbuf_v[pl.ds(i - base8, 16)][0]
            cond = jnp.logical_and(lane == 15, v < kk[15])
            nk = jnp.where(cond, v, kk)
            ni = jnp.where(cond, i, ii)
            snk, sni = lax.sort([nk, ni], dimension=0, num_keys=1)
            return (snk, sni)

        _, ids_fin = lax.fori_loop(st, en, node_body, (keys0, ids0))

        @pl.when(s >= 2)
        def _():
            pltpu.make_async_copy(rows_v.at[slot],
                                  out_hbm.at[pl.ds((sg - 2) * 16, 16)], sem_o).wait()

        @pl.when(s >= 1)
        def _():
            pltpu.make_async_copy(cs_hbm.at[ids_v.at[1 - slot]],
                                  rows_v.at[1 - slot], sem_g).wait()
            pltpu.async_copy(rows_v.at[1 - slot],
                             out_hbm.at[pl.ds((sg - 1) * 16, 16)], sem_o)

        ids_v[slot] = ids_fin
        pltpu.async_copy(cs_hbm.at[ids_v.at[slot]], rows_v.at[slot], sem_g)
        return carry

    lax.fori_loop(0, SEGS_PER_W, seg_body, 0)
    last = SEGS_PER_W - 1
    lslot = last % 2
    pltpu.make_async_copy(rows_v.at[1 - lslot],
                          out_hbm.at[pl.ds((w * SEGS_PER_W + last - 1) * 16, 16)],
                          sem_o).wait()
    pltpu.make_async_copy(cs_hbm.at[ids_v.at[lslot]], rows_v.at[lslot], sem_g).wait()
    pltpu.sync_copy(rows_v.at[lslot],
                    out_hbm.at[pl.ds((w * SEGS_PER_W + last) * 16, 16)])


@functools.lru_cache(maxsize=None)
def _sortpool_fn():
    return pl.kernel(
        _sortpool_body,
        out_type=jax.ShapeDtypeStruct((NSUB_PAD * 16, D2), jnp.float32),
        mesh=plsc.VectorSubcoreMesh(core_axis_name="c", subcore_axis_name="s"),
        compiler_params=pltpu.CompilerParams(use_tc_tiling_on_sc=False,
                                             needs_layout_passes=False),
        scratch_types=[
            pltpu.VMEM((64,), jnp.int32),
            pltpu.VMEM((VCOL_LEN,), jnp.float32),
            pltpu.VMEM((2, 16), jnp.int32),
            pltpu.VMEM((2, 16, D2), jnp.float32),
            pltpu.SemaphoreType.DMA,
            pltpu.SemaphoreType.DMA,
        ],
    )


def _sortpool(cs, vcol, starts):
    return _sortpool_fn()(cs, vcol, starts)


def _tc_head_body(d_ref, w1_ref, bc1_ref, w2_ref, bc2_ref, wl1_ref, bl1_ref,
                  wl2_ref, bl2_ref, s2g_ref, out_ref):
    z = jnp.dot(d_ref[...], w1_ref[...], preferred_element_type=jnp.float32)
    z = jax.nn.relu(z + bc1_ref[...][None, :])
    zm = z.reshape(NSUB_PAD * 8, 2, 16).max(axis=1)
    zm = zm.reshape(NSUB_PAD, 8, 16)
    zc = jnp.concatenate([zm[:, t:t + 4, :] for t in range(5)], axis=2)
    z2 = jnp.dot(zc.reshape(NSUB_PAD * 4, 80), w2_ref[...],
                 preferred_element_type=jnp.float32)
    z2 = jax.nn.relu(z2 + bc2_ref[...][None, :])
    z2v = z2.reshape(NSUB_PAD, 4, 32)
    g_iota = lax.broadcasted_iota(jnp.int32, (N_GRAPH, NSUB_PAD), 0)
    m = (s2g_ref[...][None, :] == g_iota).astype(jnp.float32)
    blocks = [jnp.dot(m, z2v[:, p, :], preferred_element_type=jnp.float32)
              for p in range(4)]
    sums = jnp.concatenate(blocks, axis=1)
    cnt = jnp.sum(m, axis=1, keepdims=True)
    g = sums / jnp.maximum(cnt, 1.0)
    g = jax.nn.relu(jnp.dot(g, wl1_ref[...], preferred_element_type=jnp.float32)
                    + bl1_ref[...][None, :])
    o = jnp.dot(g, wl2_ref[...], preferred_element_type=jnp.float32) + bl2_ref[...][None, :]
    mx = jnp.max(o, axis=-1, keepdims=True)
    lse = mx + jnp.log(jnp.sum(jnp.exp(o - mx), axis=-1, keepdims=True))
    out_ref[...] = o - lse


def _tc_head(dense, W1m, bc1, W2m, bc2, Wl1p, bl1, Wl2, bl2, s2g_pad):
    return pl.pallas_call(
        _tc_head_body,
        out_shape=jax.ShapeDtypeStruct((N_GRAPH, NUM_CLASSES), jnp.float32),
    )(dense, W1m, bc1, W2m, bc2, Wl1p, bl1, Wl2, bl2, s2g_pad)


def kernel(x, edge_index, node_to_subgraph, subgraph_to_graph,
           W0, b0, W1, b1, W2, b2, W3, b3,
           Wc1, bc1, Wc2, bc2, Wl1, bl1, Wl2, bl2):
    src, dst = edge_index[0], edge_index[1]
    pad_e = PAD_NODE + jnp.arange(E_PAD - N_EDGES, dtype=jnp.int32) % (NPAD - N_NODES)
    src_p = jnp.concatenate([src, pad_e]).reshape(NW * CPW, CHUNK)
    dst_p = jnp.concatenate([dst, pad_e]).reshape(NW * CPW, CHUNK)
    x_p = jnp.pad(x, ((0, NPAD - N_NODES), (0, 0)))

    n2s_pad = jnp.concatenate([node_to_subgraph,
                               jnp.full((NPAD - N_NODES,), 2000, jnp.int32)])
    deg_parts = _deg(dst_p)
    dinv, u, starts = _tc_pre(x_p, deg_parts, W0, n2s_pad)

    W3p = jnp.pad(W3, ((0, 0), (0, HIDDEN - 1)))
    b3p = jnp.pad(b3, (0, HIDDEN - 1))
    hs = []
    for b, Wn in ((b0, W1), (b1, W2), (b2, W3p)):
        parts = _msgpass(src_p, dst_p, u)
        h, u = _tc_layer(parts, u, dinv, b, Wn)
        hs.append(h)
    parts = _msgpass(src_p, dst_p, u)
    cs, vcol = _tc_cs(parts, u, dinv, b3p, hs[0], hs[1], hs[2])

    dense = _sortpool(cs, vcol, starts)

    W1m = jnp.pad(Wc1[:, 0, :].T, ((0, D2 - TOTAL_LATENT), (0, 0)))
    W2m = Wc2.transpose(2, 1, 0).reshape(80, 32)
    f = jnp.arange(128)
    perm = (f % 32) * 4 + f // 32
    Wl1p = Wl1[perm, :]
    s2g_pad = jnp.concatenate([subgraph_to_graph,
                               jnp.full((NSUB_PAD - N_SUB,), 1000, jnp.int32)])
    return _tc_head(dense, W1m, bc1, W2m, bc2, Wl1p, bl1, Wl2, bl2, s2g_pad)


# speedup vs baseline: 34.5075x; 1.0209x over previous
import functools

import jax
import jax.numpy as jnp
from jax import lax
from jax.experimental import pallas as pl
from jax.experimental.pallas import tpu as pltpu
from jax.experimental.pallas import tpu_sc as plsc

N_NODES = 10000
N_EDGES = 320000
N_SUB = 1000
N_GRAPH = 100
D_FEAT = 128
HIDDEN = 32
K = 16
TOTAL_LATENT = 97
NUM_CLASSES = 10

NPAD = 10240
NBUF = 6
PAD_NODE = N_NODES
NW = 32
CHUNK = 128
CPW = 80
E_PAD = NW * CPW * CHUNK
ROWS_PER_TILE = NPAD // 16


def _msgpass_body(src_hbm, dst_hbm, u_hbm, out_hbm, src_v, dst_v, rows_v, zbuf_v, acc_sh, sem_g, sem_s):
    cid = lax.axis_index("c")
    sid = lax.axis_index("s")
    wid = cid * 16 + sid
    pltpu.async_copy(src_hbm.at[pl.ds(wid * CPW, CPW)], src_v, sem_g)
    pltpu.async_copy(dst_hbm.at[pl.ds(wid * CPW, CPW)], dst_v, sem_g)

    def zf(i, c):
        zbuf_v[i // 2, pl.ds((i % 2) * 16, 16)] = jnp.zeros((16,), jnp.float32)
        return c

    lax.fori_loop(0, 2 * CHUNK, zf, 0)
    for j in range(ROWS_PER_TILE // CHUNK):
        pltpu.async_copy(zbuf_v, acc_sh.at[pl.ds(sid * ROWS_PER_TILE + j * CHUNK, CHUNK)], sem_s)
    pltpu.make_async_copy(src_hbm.at[pl.ds(wid * CPW, CPW)], src_v, sem_g).wait()
    pltpu.make_async_copy(dst_hbm.at[pl.ds(wid * CPW, CPW)], dst_v, sem_g).wait()
    for j in range(ROWS_PER_TILE // CHUNK):
        pltpu.make_async_copy(zbuf_v, acc_sh.at[pl.ds(sid * ROWS_PER_TILE + j * CHUNK, CHUNK)], sem_s).wait()
    plsc.subcore_barrier()

    pltpu.async_copy(u_hbm.at[src_v.at[0]], rows_v.at[0], sem_g)
    pltpu.async_copy(u_hbm.at[src_v.at[1]], rows_v.at[1], sem_g)

    def body(c, carry):
        slot = lax.rem(c, NBUF)
        pltpu.make_async_copy(u_hbm.at[src_v.at[c]], rows_v.at[slot], sem_g).wait()
        pltpu.async_copy(rows_v.at[slot], acc_sh.at[dst_v.at[c]], sem_s, add=True)

        @pl.when(c + 2 < CPW)
        def _():
            @pl.when(c >= NBUF - 2)
            def _():
                old = c - (NBUF - 2)
                pltpu.make_async_copy(rows_v.at[lax.rem(old, NBUF)],
                                      acc_sh.at[dst_v.at[old]], sem_s).wait()
            pltpu.async_copy(u_hbm.at[src_v.at[c + 2]],
                             rows_v.at[lax.rem(c + 2, NBUF)], sem_g)
        return carry

    lax.fori_loop(0, CPW, body, 0)

    def drain(j, carry):
        pltpu.make_async_copy(rows_v.at[lax.rem(j, NBUF)],
                              acc_sh.at[dst_v.at[j]], sem_s).wait()
        return carry

    lax.fori_loop(CPW - NBUF, CPW, drain, 0)
    plsc.subcore_barrier()
    pltpu.sync_copy(acc_sh.at[pl.ds(sid * ROWS_PER_TILE, ROWS_PER_TILE)],
                    out_hbm.at[cid, pl.ds(sid * ROWS_PER_TILE, ROWS_PER_TILE)])


@functools.lru_cache(maxsize=None)
def _msgpass_fn():
    return pl.kernel(
        _msgpass_body,
        out_type=jax.ShapeDtypeStruct((2, NPAD, HIDDEN), jnp.float32),
        mesh=plsc.VectorSubcoreMesh(core_axis_name="c", subcore_axis_name="s"),
        compiler_params=pltpu.CompilerParams(use_tc_tiling_on_sc=False),
        scratch_types=[
            pltpu.VMEM((CPW, CHUNK), jnp.int32),
            pltpu.VMEM((CPW, CHUNK), jnp.int32),
            pltpu.VMEM((NBUF, CHUNK, HIDDEN), jnp.float32),
            pltpu.VMEM((CHUNK, HIDDEN), jnp.float32),
            pltpu.VMEM_SHARED((NPAD, HIDDEN), jnp.float32),
            pltpu.SemaphoreType.DMA,
            pltpu.SemaphoreType.DMA,
        ],
    )


def _msgpass(src_p, dst_p, u):
    return _msgpass_fn()(src_p, dst_p, u)


def _deg_body(dst_hbm, out_hbm, dst_v, ones_v, zbuf_v, acc_sh):
    cid = lax.axis_index("c")
    sid = lax.axis_index("s")
    wid = cid * 16 + sid
    pltpu.sync_copy(dst_hbm.at[pl.ds(wid * CPW, CPW)], dst_v)

    def zf(i, c):
        zbuf_v[pl.ds(i * 16, 16)] = jnp.zeros((16,), jnp.float32)
        ones_v[pl.ds(i * 16, 16)] = jnp.ones((16,), jnp.float32)
        return c

    lax.fori_loop(0, CHUNK // 16, zf, 0)
    for j in range(ROWS_PER_TILE // CHUNK):
        pltpu.sync_copy(zbuf_v, acc_sh.at[pl.ds(sid * ROWS_PER_TILE + j * CHUNK, CHUNK)])
    plsc.subcore_barrier()

    def body(c, carry):
        pltpu.sync_copy(ones_v, acc_sh.at[dst_v.at[c]], add=True)
        return carry

    lax.fori_loop(0, CPW, body, 0)
    plsc.subcore_barrier()
    pltpu.sync_copy(acc_sh.at[pl.ds(sid * ROWS_PER_TILE, ROWS_PER_TILE)],
                    out_hbm.at[cid, pl.ds(sid * ROWS_PER_TILE, ROWS_PER_TILE)])


@functools.lru_cache(maxsize=None)
def _deg_fn():
    return pl.kernel(
        _deg_body,
        out_type=jax.ShapeDtypeStruct((2, NPAD), jnp.float32),
        mesh=plsc.VectorSubcoreMesh(core_axis_name="c", subcore_axis_name="s"),
        compiler_params=pltpu.CompilerParams(use_tc_tiling_on_sc=False),
        scratch_types=[
            pltpu.VMEM((CPW, CHUNK), jnp.int32),
            pltpu.VMEM((CHUNK,), jnp.float32),
            pltpu.VMEM((CHUNK,), jnp.float32),
            pltpu.VMEM_SHARED((NPAD,), jnp.float32),
        ],
    )


def _deg(dst_p):
    return _deg_fn()(dst_p)


NSTART = 1040


def _tc_xw0_body(x_ref, w0_ref, n2s_ref, v0_ref, starts_ref):
    v0_ref[...] = jnp.dot(x_ref[...], w0_ref[...], preferred_element_type=jnp.float32)
    s_iota = lax.broadcasted_iota(jnp.int32, (NSTART, 1), 0)
    acc = jnp.zeros((NSTART,), jnp.float32)
    for c in range(NPAD // 1024):
        chunk = n2s_ref[pl.ds(c * 1024, 1024)]
        acc = acc + jnp.sum((chunk[None, :] < s_iota).astype(jnp.float32), axis=1)
    starts_ref[...] = acc.astype(jnp.int32)


def _tc_xw0(x_p, W0, n2s_pad):
    return pl.pallas_call(
        _tc_xw0_body,
        out_shape=(jax.ShapeDtypeStruct((NPAD, HIDDEN), jnp.float32),
                   jax.ShapeDtypeStruct((NSTART,), jnp.int32)),
    )(x_p, W0, n2s_pad)


def _tc_scale_body(dp_ref, v0_ref, dinv_ref, u0_ref):
    dp = dp_ref[...]
    deg = 1.0 + (dp[0] + dp[1])[:, None]
    dinv = lax.rsqrt(deg)
    dinv_ref[...] = dinv
    u0_ref[...] = dinv * v0_ref[...]


def _tc_scale(deg_parts, v0):
    return pl.pallas_call(
        _tc_scale_body,
        out_shape=(jax.ShapeDtypeStruct((NPAD, 1), jnp.float32),
                   jax.ShapeDtypeStruct((NPAD, HIDDEN), jnp.float32)),
    )(deg_parts, v0)


def _tc_layer_body(dp_ref, u_ref, dinv_ref, b_ref, wn_ref, h_ref, un_ref):
    dp = dp_ref[...]
    dinv = dinv_ref[...]
    h = jnp.tanh(dinv * (dp[0] + dp[1] + u_ref[...]) + b_ref[...][None, :])
    h_ref[...] = h
    un_ref[...] = dinv * jnp.dot(h, wn_ref[...], preferred_element_type=jnp.float32)


def _tc_layer(parts, u, dinv, b, Wn):
    return pl.pallas_call(
        _tc_layer_body,
        out_shape=(jax.ShapeDtypeStruct((NPAD, HIDDEN), jnp.float32),
                   jax.ShapeDtypeStruct((NPAD, HIDDEN), jnp.float32)),
    )(parts, u, dinv, b, Wn)


D2 = 104
VCOL_LEN = NPAD + 512


def _tc_cs_body(dp_ref, u_ref, dinv_ref, b_ref, h0_ref, h1_ref, h2_ref,
                cs_ref, vcol_ref):
    dp = dp_ref[...]
    h3 = jnp.tanh(dinv_ref[...] * (dp[0] + dp[1] + u_ref[...]) + b_ref[...][None, :])
    row = lax.broadcasted_iota(jnp.int32, (NPAD, 1), 0)
    valid = (row < N_NODES).astype(jnp.float32)
    cs = jnp.concatenate(
        [h0_ref[...], h1_ref[...], h2_ref[...], h3[:, 0:1],
         jnp.zeros((NPAD, D2 - TOTAL_LATENT), jnp.float32)], axis=1)
    cs = cs * valid
    cs_ref[...] = cs
    vcol_ref[...] = jnp.concatenate([cs[:, TOTAL_LATENT - 1],
                                     jnp.zeros((VCOL_LEN - NPAD,), jnp.float32)])


def _tc_cs(parts, u, dinv, b, h0, h1, h2):
    return pl.pallas_call(
        _tc_cs_body,
        out_shape=(jax.ShapeDtypeStruct((NPAD, D2), jnp.float32),
                   jax.ShapeDtypeStruct((VCOL_LEN,), jnp.float32)),
    )(parts, u, dinv, b, h0, h1, h2)


SEGS_PER_W = 32
NSUB_PAD = 1024


def _sortpool_body(cs_hbm, vcol_hbm, starts_hbm, out_hbm,
                   starts_v, vbuf_v, ids_v, rows_v, sem_g, sem_o):
    cid = lax.axis_index("c")
    sid = lax.axis_index("s")
    w = cid * 16 + sid
    pltpu.sync_copy(starts_hbm.at[pl.ds(w * SEGS_PER_W, 40)], starts_v.at[pl.ds(0, 40)])
    s0 = starts_v[pl.ds(0, 16)][0]
    endw = starts_v[pl.ds(SEGS_PER_W, 16)][0]
    base8 = (s0 // 8) * 8
    nld = (endw - base8 + 511) // 512

    def ld(j, c):
        pltpu.sync_copy(vcol_hbm.at[pl.ds(base8 + j * 512, 512)],
                        vbuf_v.at[pl.ds(j * 512, 512)])
        return c

    lax.fori_loop(0, nld, ld, 0)
    lane = lax.iota(jnp.int32, 16)

    def seg_body(s, carry):
        sg = w * SEGS_PER_W + s
        slot = lax.rem(s, 2)
        stv = starts_v[pl.ds(s, 16)]
        st = stv[0]
        en = stv[1]
        zr = N_NODES + lax.rem(sg, NPAD - N_NODES)
        keys0 = jnp.full((16,), 3.0, jnp.float32)
        ids0 = jnp.broadcast_to(zr, (16,))

        def node_body(i, kki):
            kk, ii = kki
            v = -vbuf_v[pl.ds(i - base8, 16)][0]
            cond = jnp.logical_and(lane == 15, v < kk[15])
            nk = jnp.where(cond, v, kk)
            ni = jnp.where(cond, i, ii)
            snk, sni = lax.sort([nk, ni], dimension=0, num_keys=1)
            return (snk, sni)

        _, ids_fin = lax.fori_loop(st, en, node_body, (keys0, ids0))

        @pl.when(s >= 2)
        def _():
            pltpu.make_async_copy(rows_v.at[slot],
                                  out_hbm.at[pl.ds((sg - 2) * 16, 16)], sem_o).wait()

        @pl.when(s >= 1)
        def _():
            pltpu.make_async_copy(cs_hbm.at[ids_v.at[1 - slot]],
                                  rows_v.at[1 - slot], sem_g).wait()
            pltpu.async_copy(rows_v.at[1 - slot],
                             out_hbm.at[pl.ds((sg - 1) * 16, 16)], sem_o)

        ids_v[slot] = ids_fin
        pltpu.async_copy(cs_hbm.at[ids_v.at[slot]], rows_v.at[slot], sem_g)
        return carry

    lax.fori_loop(0, SEGS_PER_W, seg_body, 0)
    last = SEGS_PER_W - 1
    lslot = last % 2
    pltpu.make_async_copy(rows_v.at[1 - lslot],
                          out_hbm.at[pl.ds((w * SEGS_PER_W + last - 1) * 16, 16)],
                          sem_o).wait()
    pltpu.make_async_copy(cs_hbm.at[ids_v.at[lslot]], rows_v.at[lslot], sem_g).wait()
    pltpu.sync_copy(rows_v.at[lslot],
                    out_hbm.at[pl.ds((w * SEGS_PER_W + last) * 16, 16)])


@functools.lru_cache(maxsize=None)
def _sortpool_fn():
    return pl.kernel(
        _sortpool_body,
        out_type=jax.ShapeDtypeStruct((NSUB_PAD * 16, D2), jnp.float32),
        mesh=plsc.VectorSubcoreMesh(core_axis_name="c", subcore_axis_name="s"),
        compiler_params=pltpu.CompilerParams(use_tc_tiling_on_sc=False,
                                             needs_layout_passes=False),
        scratch_types=[
            pltpu.VMEM((64,), jnp.int32),
            pltpu.VMEM((VCOL_LEN,), jnp.float32),
            pltpu.VMEM((2, 16), jnp.int32),
            pltpu.VMEM((2, 16, D2), jnp.float32),
            pltpu.SemaphoreType.DMA,
            pltpu.SemaphoreType.DMA,
        ],
    )


def _sortpool(cs, vcol, starts):
    return _sortpool_fn()(cs, vcol, starts)


def _tc_head_body(d_ref, w1_ref, bc1_ref, w2_ref, bc2_ref, wl1_ref, bl1_ref,
                  wl2_ref, bl2_ref, s2g_ref, out_ref):
    z = jnp.dot(d_ref[...], w1_ref[...], preferred_element_type=jnp.float32)
    z = jax.nn.relu(z + bc1_ref[...][None, :])
    zm = z.reshape(NSUB_PAD * 8, 2, 16).max(axis=1)
    zm = zm.reshape(NSUB_PAD, 8, 16)
    zc = jnp.concatenate([zm[:, t:t + 4, :] for t in range(5)], axis=2)
    z2 = jnp.dot(zc.reshape(NSUB_PAD * 4, 80), w2_ref[...],
                 preferred_element_type=jnp.float32)
    z2 = jax.nn.relu(z2 + bc2_ref[...][None, :])
    z2v = z2.reshape(NSUB_PAD, 4, 32)
    g_iota = lax.broadcasted_iota(jnp.int32, (N_GRAPH, NSUB_PAD), 0)
    m = (s2g_ref[...][None, :] == g_iota).astype(jnp.float32)
    blocks = [jnp.dot(m, z2v[:, p, :], preferred_element_type=jnp.float32)
              for p in range(4)]
    sums = jnp.concatenate(blocks, axis=1)
    cnt = jnp.sum(m, axis=1, keepdims=True)
    g = sums / jnp.maximum(cnt, 1.0)
    g = jax.nn.relu(jnp.dot(g, wl1_ref[...], preferred_element_type=jnp.float32)
                    + bl1_ref[...][None, :])
    o = jnp.dot(g, wl2_ref[...], preferred_element_type=jnp.float32) + bl2_ref[...][None, :]
    mx = jnp.max(o, axis=-1, keepdims=True)
    lse = mx + jnp.log(jnp.sum(jnp.exp(o - mx), axis=-1, keepdims=True))
    out_ref[...] = o - lse


def _tc_head(dense, W1m, bc1, W2m, bc2, Wl1p, bl1, Wl2, bl2, s2g_pad):
    return pl.pallas_call(
        _tc_head_body,
        out_shape=jax.ShapeDtypeStruct((N_GRAPH, NUM_CLASSES), jnp.float32),
    )(dense, W1m, bc1, W2m, bc2, Wl1p, bl1, Wl2, bl2, s2g_pad)


def kernel(x, edge_index, node_to_subgraph, subgraph_to_graph,
           W0, b0, W1, b1, W2, b2, W3, b3,
           Wc1, bc1, Wc2, bc2, Wl1, bl1, Wl2, bl2):
    src, dst = edge_index[0], edge_index[1]
    pad_e = PAD_NODE + jnp.arange(E_PAD - N_EDGES, dtype=jnp.int32) % (NPAD - N_NODES)
    src_p = jnp.concatenate([src, pad_e]).reshape(NW * CPW, CHUNK)
    dst_p = jnp.concatenate([dst, pad_e]).reshape(NW * CPW, CHUNK)
    x_p = jnp.pad(x, ((0, NPAD - N_NODES), (0, 0)))

    n2s_pad = jnp.concatenate([node_to_subgraph,
                               jnp.full((NPAD - N_NODES,), 2000, jnp.int32)])
    v0, starts = _tc_xw0(x_p, W0, n2s_pad)
    deg_parts = _deg(dst_p)
    dinv, u = _tc_scale(deg_parts, v0)

    W3p = jnp.pad(W3, ((0, 0), (0, HIDDEN - 1)))
    b3p = jnp.pad(b3, (0, HIDDEN - 1))
    hs = []
    for b, Wn in ((b0, W1), (b1, W2), (b2, W3p)):
        parts = _msgpass(src_p, dst_p, u)
        h, u = _tc_layer(parts, u, dinv, b, Wn)
        hs.append(h)
    parts = _msgpass(src_p, dst_p, u)
    cs, vcol = _tc_cs(parts, u, dinv, b3p, hs[0], hs[1], hs[2])

    dense = _sortpool(cs, vcol, starts)

    W1m = jnp.pad(Wc1[:, 0, :].T, ((0, D2 - TOTAL_LATENT), (0, 0)))
    W2m = Wc2.transpose(2, 1, 0).reshape(80, 32)
    f = jnp.arange(128)
    perm = (f % 32) * 4 + f // 32
    Wl1p = Wl1[perm, :]
    s2g_pad = jnp.concatenate([subgraph_to_graph,
                               jnp.full((NSUB_PAD - N_SUB,), 1000, jnp.int32)])
    return _tc_head(dense, W1m, bc1, W2m, bc2, Wl1p, bl1, Wl2, bl2, s2g_pad)


# speedup vs baseline: 35.8847x; 1.0399x over previous
import functools

import jax
import jax.numpy as jnp
from jax import lax
from jax.experimental import pallas as pl
from jax.experimental.pallas import tpu as pltpu
from jax.experimental.pallas import tpu_sc as plsc

N_NODES = 10000
N_EDGES = 320000
N_SUB = 1000
N_GRAPH = 100
D_FEAT = 128
HIDDEN = 32
K = 16
TOTAL_LATENT = 97
NUM_CLASSES = 10

NPAD = 10240
NBUF = 6
PAD_NODE = N_NODES
NW = 32
CHUNK = 128
CPW = 80
E_PAD = NW * CPW * CHUNK
ROWS_PER_TILE = NPAD // 16


def _msgpass_body(src_hbm, dst_hbm, u_hbm, out_hbm, src_v, dst_v, rows_v, zbuf_v, acc_sh, sem_g, sem_s):
    cid = lax.axis_index("c")
    sid = lax.axis_index("s")
    wid = cid * 16 + sid
    pltpu.async_copy(src_hbm.at[pl.ds(wid * CPW, CPW)], src_v, sem_g)
    pltpu.async_copy(dst_hbm.at[pl.ds(wid * CPW, CPW)], dst_v, sem_g)

    def zf(i, c):
        zbuf_v[i // 2, pl.ds((i % 2) * 16, 16)] = jnp.zeros((16,), jnp.float32)
        return c

    lax.fori_loop(0, 2 * CHUNK, zf, 0)
    for j in range(ROWS_PER_TILE // CHUNK):
        pltpu.async_copy(zbuf_v, acc_sh.at[pl.ds(sid * ROWS_PER_TILE + j * CHUNK, CHUNK)], sem_s)
    pltpu.make_async_copy(src_hbm.at[pl.ds(wid * CPW, CPW)], src_v, sem_g).wait()
    pltpu.make_async_copy(dst_hbm.at[pl.ds(wid * CPW, CPW)], dst_v, sem_g).wait()
    for j in range(ROWS_PER_TILE // CHUNK):
        pltpu.make_async_copy(zbuf_v, acc_sh.at[pl.ds(sid * ROWS_PER_TILE + j * CHUNK, CHUNK)], sem_s).wait()
    plsc.subcore_barrier()

    pltpu.async_copy(u_hbm.at[src_v.at[0]], rows_v.at[0], sem_g)
    pltpu.async_copy(u_hbm.at[src_v.at[1]], rows_v.at[1], sem_g)

    def body(c, carry):
        slot = lax.rem(c, NBUF)
        pltpu.make_async_copy(u_hbm.at[src_v.at[c]], rows_v.at[slot], sem_g).wait()
        pltpu.async_copy(rows_v.at[slot], acc_sh.at[dst_v.at[c]], sem_s, add=True)

        @pl.when(c + 2 < CPW)
        def _():
            @pl.when(c >= NBUF - 2)
            def _():
                old = c - (NBUF - 2)
                pltpu.make_async_copy(rows_v.at[lax.rem(old, NBUF)],
                                      acc_sh.at[dst_v.at[old]], sem_s).wait()
            pltpu.async_copy(u_hbm.at[src_v.at[c + 2]],
                             rows_v.at[lax.rem(c + 2, NBUF)], sem_g)
        return carry

    lax.fori_loop(0, CPW, body, 0)

    def drain(j, carry):
        pltpu.make_async_copy(rows_v.at[lax.rem(j, NBUF)],
                              acc_sh.at[dst_v.at[j]], sem_s).wait()
        return carry

    lax.fori_loop(CPW - NBUF, CPW, drain, 0)
    plsc.subcore_barrier()
    pltpu.sync_copy(acc_sh.at[pl.ds(sid * ROWS_PER_TILE, ROWS_PER_TILE)],
                    out_hbm.at[cid, pl.ds(sid * ROWS_PER_TILE, ROWS_PER_TILE)])


@functools.lru_cache(maxsize=None)
def _msgpass_fn():
    return pl.kernel(
        _msgpass_body,
        out_type=jax.ShapeDtypeStruct((2, NPAD, HIDDEN), jnp.float32),
        mesh=plsc.VectorSubcoreMesh(core_axis_name="c", subcore_axis_name="s"),
        compiler_params=pltpu.CompilerParams(use_tc_tiling_on_sc=False),
        scratch_types=[
            pltpu.VMEM((CPW, CHUNK), jnp.int32),
            pltpu.VMEM((CPW, CHUNK), jnp.int32),
            pltpu.VMEM((NBUF, CHUNK, HIDDEN), jnp.float32),
            pltpu.VMEM((CHUNK, HIDDEN), jnp.float32),
            pltpu.VMEM_SHARED((NPAD, HIDDEN), jnp.float32),
            pltpu.SemaphoreType.DMA,
            pltpu.SemaphoreType.DMA,
        ],
    )


def _msgpass(src_p, dst_p, u):
    return _msgpass_fn()(src_p, dst_p, u)


def _deg_body(dst_hbm, out_hbm, dst_v, ones_v, zbuf_v, acc_sh):
    cid = lax.axis_index("c")
    sid = lax.axis_index("s")
    wid = cid * 16 + sid
    pltpu.sync_copy(dst_hbm.at[pl.ds(wid * CPW, CPW)], dst_v)

    def zf(i, c):
        zbuf_v[pl.ds(i * 16, 16)] = jnp.zeros((16,), jnp.float32)
        ones_v[pl.ds(i * 16, 16)] = jnp.ones((16,), jnp.float32)
        return c

    lax.fori_loop(0, CHUNK // 16, zf, 0)
    for j in range(ROWS_PER_TILE // CHUNK):
        pltpu.sync_copy(zbuf_v, acc_sh.at[pl.ds(sid * ROWS_PER_TILE + j * CHUNK, CHUNK)])
    plsc.subcore_barrier()

    def body(c, carry):
        pltpu.sync_copy(ones_v, acc_sh.at[dst_v.at[c]], add=True)
        return carry

    lax.fori_loop(0, CPW, body, 0)
    plsc.subcore_barrier()
    pltpu.sync_copy(acc_sh.at[pl.ds(sid * ROWS_PER_TILE, ROWS_PER_TILE)],
                    out_hbm.at[cid, pl.ds(sid * ROWS_PER_TILE, ROWS_PER_TILE)])


@functools.lru_cache(maxsize=None)
def _deg_fn():
    return pl.kernel(
        _deg_body,
        out_type=jax.ShapeDtypeStruct((2, NPAD), jnp.float32),
        mesh=plsc.VectorSubcoreMesh(core_axis_name="c", subcore_axis_name="s"),
        compiler_params=pltpu.CompilerParams(use_tc_tiling_on_sc=False),
        scratch_types=[
            pltpu.VMEM((CPW, CHUNK), jnp.int32),
            pltpu.VMEM((CHUNK,), jnp.float32),
            pltpu.VMEM((CHUNK,), jnp.float32),
            pltpu.VMEM_SHARED((NPAD,), jnp.float32),
        ],
    )


def _deg(dst_p):
    return _deg_fn()(dst_p)


NSTART = 1040


def _tc_xw0_body(x_ref, w0_ref, n2s_ref, v0_ref, starts_ref):
    v0_ref[...] = jnp.dot(x_ref[...], w0_ref[...], preferred_element_type=jnp.float32)
    s_iota = lax.broadcasted_iota(jnp.int32, (NSTART, 1), 0)
    acc = jnp.zeros((NSTART,), jnp.float32)
    for c in range(NPAD // 1024):
        chunk = n2s_ref[pl.ds(c * 1024, 1024)]
        acc = acc + jnp.sum((chunk[None, :] < s_iota).astype(jnp.float32), axis=1)
    starts_ref[...] = acc.astype(jnp.int32)


def _tc_xw0(x_p, W0, n2s_pad):
    return pl.pallas_call(
        _tc_xw0_body,
        out_shape=(jax.ShapeDtypeStruct((NPAD, HIDDEN), jnp.float32),
                   jax.ShapeDtypeStruct((NSTART,), jnp.int32)),
    )(x_p, W0, n2s_pad)


def _tc_scale_body(dp_ref, v0_ref, dinv_ref, u0_ref):
    dp = dp_ref[...]
    deg = 1.0 + (dp[0] + dp[1])[:, None]
    dinv = lax.rsqrt(deg)
    dinv_ref[...] = dinv
    u0_ref[...] = dinv * v0_ref[...]


def _tc_scale(deg_parts, v0):
    return pl.pallas_call(
        _tc_scale_body,
        out_shape=(jax.ShapeDtypeStruct((NPAD, 1), jnp.float32),
                   jax.ShapeDtypeStruct((NPAD, HIDDEN), jnp.float32)),
    )(deg_parts, v0)


def _tc_layer_body(dp_ref, u_ref, dinv_ref, b_ref, wn_ref, h_ref, un_ref):
    dp = dp_ref[...]
    dinv = dinv_ref[...]
    h = jnp.tanh(dinv * (dp[0] + dp[1] + u_ref[...]) + b_ref[...][None, :])
    h_ref[...] = h
    un_ref[...] = dinv * jnp.dot(h, wn_ref[...], preferred_element_type=jnp.float32)


def _tc_layer(parts, u, dinv, b, Wn):
    return pl.pallas_call(
        _tc_layer_body,
        out_shape=(jax.ShapeDtypeStruct((NPAD, HIDDEN), jnp.float32),
                   jax.ShapeDtypeStruct((NPAD, HIDDEN), jnp.float32)),
    )(parts, u, dinv, b, Wn)


D2 = 104
VCOL_LEN = NPAD + 512


def _tc_cs_body(dp_ref, u_ref, dinv_ref, b_ref, h0_ref, h1_ref, h2_ref,
                cs_ref, vcol_ref):
    dp = dp_ref[...]
    h3 = jnp.tanh(dinv_ref[...] * (dp[0] + dp[1] + u_ref[...]) + b_ref[...][None, :])
    row = lax.broadcasted_iota(jnp.int32, (NPAD, 1), 0)
    valid = (row < N_NODES).astype(jnp.float32)
    cs = jnp.concatenate(
        [h0_ref[...], h1_ref[...], h2_ref[...], h3[:, 0:1],
         jnp.zeros((NPAD, D2 - TOTAL_LATENT), jnp.float32)], axis=1)
    cs = cs * valid
    cs_ref[...] = cs
    vcol_ref[...] = jnp.concatenate([cs[:, TOTAL_LATENT - 1],
                                     jnp.zeros((VCOL_LEN - NPAD,), jnp.float32)])


def _tc_cs(parts, u, dinv, b, h0, h1, h2):
    return pl.pallas_call(
        _tc_cs_body,
        out_shape=(jax.ShapeDtypeStruct((NPAD, D2), jnp.float32),
                   jax.ShapeDtypeStruct((VCOL_LEN,), jnp.float32)),
    )(parts, u, dinv, b, h0, h1, h2)


SEGS_PER_W = 32
NSUB_PAD = 1024
SPR = 4


def _sortpool_body(cs_hbm, vcol_hbm, starts_hbm, out_hbm,
                   starts_v, vbuf_v, ids_v, rows_v, sem_g, sem_o):
    cid = lax.axis_index("c")
    sid = lax.axis_index("s")
    w = cid * 16 + sid
    pltpu.sync_copy(starts_hbm.at[pl.ds(w * SEGS_PER_W, 40)], starts_v.at[pl.ds(0, 40)])
    s0 = starts_v[pl.ds(0, 16)][0]
    endw = starts_v[pl.ds(SEGS_PER_W, 16)][0]
    base8 = (s0 // 8) * 8
    nld = (endw - base8 + 511) // 512

    def ld(j, c):
        pltpu.sync_copy(vcol_hbm.at[pl.ds(base8 + j * 512, 512)],
                        vbuf_v.at[pl.ds(j * 512, 512)])
        return c

    lax.fori_loop(0, nld, ld, 0)
    lane = lax.iota(jnp.int32, 16)

    def seg_body(s, carry):
        sg = w * SEGS_PER_W + s
        slot = lax.rem(s, SPR)
        stv = starts_v[pl.ds(s, 16)]
        st = stv[0]
        en = stv[1]
        zr = N_NODES + lax.rem(sg, NPAD - N_NODES)
        keys0 = jnp.full((16,), 3.0, jnp.float32)
        ids0 = jnp.broadcast_to(zr, (16,))

        def node_body(i, kki):
            kk, ii = kki
            v = -vbuf_v[pl.ds(i - base8, 16)][0]
            cond = jnp.logical_and(lane == 15, v < kk[15])
            nk = jnp.where(cond, v, kk)
            ni = jnp.where(cond, i, ii)
            snk, sni = lax.sort([nk, ni], dimension=0, num_keys=1)
            return (snk, sni)

        _, ids_fin = lax.fori_loop(st, en, node_body, (keys0, ids0))

        @pl.when(s >= SPR)
        def _():
            old = s - SPR
            pltpu.make_async_copy(rows_v.at[lax.rem(old, SPR)],
                                  out_hbm.at[pl.ds((w * SEGS_PER_W + old) * 16, 16)],
                                  sem_o).wait()

        @pl.when(s >= SPR - 1)
        def _():
            mid = s - (SPR - 1)
            mslot = lax.rem(mid, SPR)
            pltpu.make_async_copy(cs_hbm.at[ids_v.at[mslot]],
                                  rows_v.at[mslot], sem_g).wait()
            pltpu.async_copy(rows_v.at[mslot],
                             out_hbm.at[pl.ds((w * SEGS_PER_W + mid) * 16, 16)], sem_o)

        ids_v[slot] = ids_fin
        pltpu.async_copy(cs_hbm.at[ids_v.at[slot]], rows_v.at[slot], sem_g)
        return carry

    lax.fori_loop(0, SEGS_PER_W, seg_body, 0)

    def fin_gather(j, carry):
        jslot = lax.rem(j, SPR)
        pltpu.make_async_copy(cs_hbm.at[ids_v.at[jslot]], rows_v.at[jslot], sem_g).wait()
        pltpu.async_copy(rows_v.at[jslot],
                         out_hbm.at[pl.ds((w * SEGS_PER_W + j) * 16, 16)], sem_o)
        return carry

    lax.fori_loop(SEGS_PER_W - (SPR - 1), SEGS_PER_W, fin_gather, 0)

    def fin_store(j, carry):
        pltpu.make_async_copy(rows_v.at[lax.rem(j, SPR)],
                              out_hbm.at[pl.ds((w * SEGS_PER_W + j) * 16, 16)],
                              sem_o).wait()
        return carry

    lax.fori_loop(SEGS_PER_W - SPR, SEGS_PER_W, fin_store, 0)


@functools.lru_cache(maxsize=None)
def _sortpool_fn():
    return pl.kernel(
        _sortpool_body,
        out_type=jax.ShapeDtypeStruct((NSUB_PAD * 16, D2), jnp.float32),
        mesh=plsc.VectorSubcoreMesh(core_axis_name="c", subcore_axis_name="s"),
        compiler_params=pltpu.CompilerParams(use_tc_tiling_on_sc=False,
                                             needs_layout_passes=False),
        scratch_types=[
            pltpu.VMEM((64,), jnp.int32),
            pltpu.VMEM((VCOL_LEN,), jnp.float32),
            pltpu.VMEM((SPR, 16), jnp.int32),
            pltpu.VMEM((SPR, 16, D2), jnp.float32),
            pltpu.SemaphoreType.DMA,
            pltpu.SemaphoreType.DMA,
        ],
    )


def _sortpool(cs, vcol, starts):
    return _sortpool_fn()(cs, vcol, starts)


def _tc_head_body(d_ref, w1_ref, bc1_ref, w2_ref, bc2_ref, wl1_ref, bl1_ref,
                  wl2_ref, bl2_ref, s2g_ref, out_ref):
    z = jnp.dot(d_ref[...], w1_ref[...], preferred_element_type=jnp.float32)
    z = jax.nn.relu(z + bc1_ref[...][None, :])
    zm = z.reshape(NSUB_PAD * 8, 2, 16).max(axis=1)
    zm = zm.reshape(NSUB_PAD, 8, 16)
    zc = jnp.concatenate([zm[:, t:t + 4, :] for t in range(5)], axis=2)
    z2 = jnp.dot(zc.reshape(NSUB_PAD * 4, 80), w2_ref[...],
                 preferred_element_type=jnp.float32)
    z2 = jax.nn.relu(z2 + bc2_ref[...][None, :])
    z2v = z2.reshape(NSUB_PAD, 4, 32)
    g_iota = lax.broadcasted_iota(jnp.int32, (N_GRAPH, NSUB_PAD), 0)
    m = (s2g_ref[...][None, :] == g_iota).astype(jnp.float32)
    blocks = [jnp.dot(m, z2v[:, p, :], preferred_element_type=jnp.float32)
              for p in range(4)]
    sums = jnp.concatenate(blocks, axis=1)
    cnt = jnp.sum(m, axis=1, keepdims=True)
    g = sums / jnp.maximum(cnt, 1.0)
    g = jax.nn.relu(jnp.dot(g, wl1_ref[...], preferred_element_type=jnp.float32)
                    + bl1_ref[...][None, :])
    o = jnp.dot(g, wl2_ref[...], preferred_element_type=jnp.float32) + bl2_ref[...][None, :]
    mx = jnp.max(o, axis=-1, keepdims=True)
    lse = mx + jnp.log(jnp.sum(jnp.exp(o - mx), axis=-1, keepdims=True))
    out_ref[...] = o - lse


def _tc_head(dense, W1m, bc1, W2m, bc2, Wl1p, bl1, Wl2, bl2, s2g_pad):
    return pl.pallas_call(
        _tc_head_body,
        out_shape=jax.ShapeDtypeStruct((N_GRAPH, NUM_CLASSES), jnp.float32),
    )(dense, W1m, bc1, W2m, bc2, Wl1p, bl1, Wl2, bl2, s2g_pad)


def kernel(x, edge_index, node_to_subgraph, subgraph_to_graph,
           W0, b0, W1, b1, W2, b2, W3, b3,
           Wc1, bc1, Wc2, bc2, Wl1, bl1, Wl2, bl2):
    src, dst = edge_index[0], edge_index[1]
    pad_e = PAD_NODE + jnp.arange(E_PAD - N_EDGES, dtype=jnp.int32) % (NPAD - N_NODES)
    src_p = jnp.concatenate([src, pad_e]).reshape(NW * CPW, CHUNK)
    dst_p = jnp.concatenate([dst, pad_e]).reshape(NW * CPW, CHUNK)
    x_p = jnp.pad(x, ((0, NPAD - N_NODES), (0, 0)))

    n2s_pad = jnp.concatenate([node_to_subgraph,
                               jnp.full((NPAD - N_NODES,), 2000, jnp.int32)])
    v0, starts = _tc_xw0(x_p, W0, n2s_pad)
    deg_parts = _deg(dst_p)
    dinv, u = _tc_scale(deg_parts, v0)

    W3p = jnp.pad(W3, ((0, 0), (0, HIDDEN - 1)))
    b3p = jnp.pad(b3, (0, HIDDEN - 1))
    hs = []
    for b, Wn in ((b0, W1), (b1, W2), (b2, W3p)):
        parts = _msgpass(src_p, dst_p, u)
        h, u = _tc_layer(parts, u, dinv, b, Wn)
        hs.append(h)
    parts = _msgpass(src_p, dst_p, u)
    cs, vcol = _tc_cs(parts, u, dinv, b3p, hs[0], hs[1], hs[2])

    dense = _sortpool(cs, vcol, starts)

    W1m = jnp.pad(Wc1[:, 0, :].T, ((0, D2 - TOTAL_LATENT), (0, 0)))
    W2m = Wc2.transpose(2, 1, 0).reshape(80, 32)
    f = jnp.arange(128)
    perm = (f % 32) * 4 + f // 32
    Wl1p = Wl1[perm, :]
    s2g_pad = jnp.concatenate([subgraph_to_graph,
                               jnp.full((NSUB_PAD - N_SUB,), 1000, jnp.int32)])
    return _tc_head(dense, W1m, bc1, W2m, bc2, Wl1p, bl1, Wl2, bl2, s2g_pad)


# speedup vs baseline: 39.2011x; 1.0924x over previous
import functools

import jax
import jax.numpy as jnp
from jax import lax
from jax.experimental import pallas as pl
from jax.experimental.pallas import tpu as pltpu
from jax.experimental.pallas import tpu_sc as plsc

N_NODES = 10000
N_EDGES = 320000
N_SUB = 1000
N_GRAPH = 100
D_FEAT = 128
HIDDEN = 32
K = 16
TOTAL_LATENT = 97
NUM_CLASSES = 10

NPAD = 10240
NBUF = 8
PAD_NODE = N_NODES
NW = 32
CHUNK = 128
CPW = 80
E_PAD = NW * CPW * CHUNK
ROWS_PER_TILE = NPAD // 16


def _msgpass_body(src_hbm, dst_hbm, u_hbm, out_hbm, src_v, dst_v, rows_v, zbuf_v, acc_sh, sem_g, sem_s):
    cid = lax.axis_index("c")
    sid = lax.axis_index("s")
    wid = cid * 16 + sid
    pltpu.async_copy(src_hbm.at[pl.ds(wid * CPW, CPW)], src_v, sem_g)
    pltpu.async_copy(dst_hbm.at[pl.ds(wid * CPW, CPW)], dst_v, sem_g)

    def zf(i, c):
        zbuf_v[i // 2, pl.ds((i % 2) * 16, 16)] = jnp.zeros((16,), jnp.float32)
        return c

    lax.fori_loop(0, 2 * CHUNK, zf, 0)
    for j in range(ROWS_PER_TILE // CHUNK):
        pltpu.async_copy(zbuf_v, acc_sh.at[pl.ds(sid * ROWS_PER_TILE + j * CHUNK, CHUNK)], sem_s)
    pltpu.make_async_copy(src_hbm.at[pl.ds(wid * CPW, CPW)], src_v, sem_g).wait()
    pltpu.make_async_copy(dst_hbm.at[pl.ds(wid * CPW, CPW)], dst_v, sem_g).wait()
    for j in range(ROWS_PER_TILE // CHUNK):
        pltpu.make_async_copy(zbuf_v, acc_sh.at[pl.ds(sid * ROWS_PER_TILE + j * CHUNK, CHUNK)], sem_s).wait()
    plsc.subcore_barrier()

    pltpu.async_copy(u_hbm.at[src_v.at[0]], rows_v.at[0], sem_g)
    pltpu.async_copy(u_hbm.at[src_v.at[1]], rows_v.at[1], sem_g)
    pltpu.async_copy(u_hbm.at[src_v.at[2]], rows_v.at[2], sem_g)

    def body(c, carry):
        slot = lax.rem(c, NBUF)
        pltpu.make_async_copy(u_hbm.at[src_v.at[c]], rows_v.at[slot], sem_g).wait()
        pltpu.async_copy(rows_v.at[slot], acc_sh.at[dst_v.at[c]], sem_s, add=True)

        @pl.when(c + 3 < CPW)
        def _():
            @pl.when(c >= NBUF - 3)
            def _():
                old = c - (NBUF - 3)
                pltpu.make_async_copy(rows_v.at[lax.rem(old, NBUF)],
                                      acc_sh.at[dst_v.at[old]], sem_s).wait()
            pltpu.async_copy(u_hbm.at[src_v.at[c + 3]],
                             rows_v.at[lax.rem(c + 3, NBUF)], sem_g)
        return carry

    lax.fori_loop(0, CPW, body, 0)

    def drain(j, carry):
        pltpu.make_async_copy(rows_v.at[lax.rem(j, NBUF)],
                              acc_sh.at[dst_v.at[j]], sem_s).wait()
        return carry

    lax.fori_loop(CPW - NBUF, CPW, drain, 0)
    plsc.subcore_barrier()
    pltpu.sync_copy(acc_sh.at[pl.ds(sid * ROWS_PER_TILE, ROWS_PER_TILE)],
                    out_hbm.at[cid, pl.ds(sid * ROWS_PER_TILE, ROWS_PER_TILE)])


@functools.lru_cache(maxsize=None)
def _msgpass_fn():
    return pl.kernel(
        _msgpass_body,
        out_type=jax.ShapeDtypeStruct((2, NPAD, HIDDEN), jnp.float32),
        mesh=plsc.VectorSubcoreMesh(core_axis_name="c", subcore_axis_name="s"),
        compiler_params=pltpu.CompilerParams(use_tc_tiling_on_sc=False),
        scratch_types=[
            pltpu.VMEM((CPW, CHUNK), jnp.int32),
            pltpu.VMEM((CPW, CHUNK), jnp.int32),
            pltpu.VMEM((NBUF, CHUNK, HIDDEN), jnp.float32),
            pltpu.VMEM((CHUNK, HIDDEN), jnp.float32),
            pltpu.VMEM_SHARED((NPAD, HIDDEN), jnp.float32),
            pltpu.SemaphoreType.DMA,
            pltpu.SemaphoreType.DMA,
        ],
    )


def _msgpass(src_p, dst_p, u):
    return _msgpass_fn()(src_p, dst_p, u)


def _deg_body(dst_hbm, out_hbm, dst_v, ones_v, zbuf_v, acc_sh):
    cid = lax.axis_index("c")
    sid = lax.axis_index("s")
    wid = cid * 16 + sid
    pltpu.sync_copy(dst_hbm.at[pl.ds(wid * CPW, CPW)], dst_v)

    def zf(i, c):
        zbuf_v[pl.ds(i * 16, 16)] = jnp.zeros((16,), jnp.float32)
        ones_v[pl.ds(i * 16, 16)] = jnp.ones((16,), jnp.float32)
        return c

    lax.fori_loop(0, CHUNK // 16, zf, 0)
    for j in range(ROWS_PER_TILE // CHUNK):
        pltpu.sync_copy(zbuf_v, acc_sh.at[pl.ds(sid * ROWS_PER_TILE + j * CHUNK, CHUNK)])
    plsc.subcore_barrier()

    def body(c, carry):
        pltpu.sync_copy(ones_v, acc_sh.at[dst_v.at[c]], add=True)
        return carry

    lax.fori_loop(0, CPW, body, 0)
    plsc.subcore_barrier()
    pltpu.sync_copy(acc_sh.at[pl.ds(sid * ROWS_PER_TILE, ROWS_PER_TILE)],
                    out_hbm.at[cid, pl.ds(sid * ROWS_PER_TILE, ROWS_PER_TILE)])


@functools.lru_cache(maxsize=None)
def _deg_fn():
    return pl.kernel(
        _deg_body,
        out_type=jax.ShapeDtypeStruct((2, NPAD), jnp.float32),
        mesh=plsc.VectorSubcoreMesh(core_axis_name="c", subcore_axis_name="s"),
        compiler_params=pltpu.CompilerParams(use_tc_tiling_on_sc=False),
        scratch_types=[
            pltpu.VMEM((CPW, CHUNK), jnp.int32),
            pltpu.VMEM((CHUNK,), jnp.float32),
            pltpu.VMEM((CHUNK,), jnp.float32),
            pltpu.VMEM_SHARED((NPAD,), jnp.float32),
        ],
    )


def _deg(dst_p):
    return _deg_fn()(dst_p)


NSTART = 1040


def _tc_xw0_body(x_ref, w0_ref, n2s_ref, v0_ref, starts_ref):
    v0_ref[...] = jnp.dot(x_ref[...], w0_ref[...], preferred_element_type=jnp.float32)
    s_iota = lax.broadcasted_iota(jnp.int32, (NSTART, 1), 0)
    acc = jnp.zeros((NSTART,), jnp.float32)
    for c in range(NPAD // 1024):
        chunk = n2s_ref[pl.ds(c * 1024, 1024)]
        acc = acc + jnp.sum((chunk[None, :] < s_iota).astype(jnp.float32), axis=1)
    starts_ref[...] = acc.astype(jnp.int32)


def _tc_xw0(x_p, W0, n2s_pad):
    return pl.pallas_call(
        _tc_xw0_body,
        out_shape=(jax.ShapeDtypeStruct((NPAD, HIDDEN), jnp.float32),
                   jax.ShapeDtypeStruct((NSTART,), jnp.int32)),
    )(x_p, W0, n2s_pad)


def _tc_scale_body(dp_ref, v0_ref, dinv_ref, u0_ref):
    dp = dp_ref[...]
    deg = 1.0 + (dp[0] + dp[1])[:, None]
    dinv = lax.rsqrt(deg)
    dinv_ref[...] = dinv
    u0_ref[...] = dinv * v0_ref[...]


def _tc_scale(deg_parts, v0):
    return pl.pallas_call(
        _tc_scale_body,
        out_shape=(jax.ShapeDtypeStruct((NPAD, 1), jnp.float32),
                   jax.ShapeDtypeStruct((NPAD, HIDDEN), jnp.float32)),
    )(deg_parts, v0)


def _tc_layer_body(dp_ref, u_ref, dinv_ref, b_ref, wn_ref, h_ref, un_ref):
    dp = dp_ref[...]
    dinv = dinv_ref[...]
    h = jnp.tanh(dinv * (dp[0] + dp[1] + u_ref[...]) + b_ref[...][None, :])
    h_ref[...] = h
    un_ref[...] = dinv * jnp.dot(h, wn_ref[...], preferred_element_type=jnp.float32)


def _tc_layer(parts, u, dinv, b, Wn):
    return pl.pallas_call(
        _tc_layer_body,
        out_shape=(jax.ShapeDtypeStruct((NPAD, HIDDEN), jnp.float32),
                   jax.ShapeDtypeStruct((NPAD, HIDDEN), jnp.float32)),
    )(parts, u, dinv, b, Wn)


D2 = 104
VCOL_LEN = NPAD + 512


def _tc_cs_body(dp_ref, u_ref, dinv_ref, b_ref, h0_ref, h1_ref, h2_ref,
                cs_ref, vcol_ref):
    dp = dp_ref[...]
    h3 = jnp.tanh(dinv_ref[...] * (dp[0] + dp[1] + u_ref[...]) + b_ref[...][None, :])
    row = lax.broadcasted_iota(jnp.int32, (NPAD, 1), 0)
    valid = (row < N_NODES).astype(jnp.float32)
    cs = jnp.concatenate(
        [h0_ref[...], h1_ref[...], h2_ref[...], h3[:, 0:1],
         jnp.zeros((NPAD, D2 - TOTAL_LATENT), jnp.float32)], axis=1)
    cs = cs * valid
    cs_ref[...] = cs
    vcol_ref[...] = jnp.concatenate([cs[:, TOTAL_LATENT - 1],
                                     jnp.zeros((VCOL_LEN - NPAD,), jnp.float32)])


def _tc_cs(parts, u, dinv, b, h0, h1, h2):
    return pl.pallas_call(
        _tc_cs_body,
        out_shape=(jax.ShapeDtypeStruct((NPAD, D2), jnp.float32),
                   jax.ShapeDtypeStruct((VCOL_LEN,), jnp.float32)),
    )(parts, u, dinv, b, h0, h1, h2)


SEGS_PER_W = 32
NSUB_PAD = 1024
SPR = 4


def _sortpool_body(cs_hbm, vcol_hbm, starts_hbm, out_hbm,
                   starts_v, vbuf_v, ids_v, rows_v, sem_g, sem_o):
    cid = lax.axis_index("c")
    sid = lax.axis_index("s")
    w = cid * 16 + sid
    pltpu.sync_copy(starts_hbm.at[pl.ds(w * SEGS_PER_W, 40)], starts_v.at[pl.ds(0, 40)])
    s0 = starts_v[pl.ds(0, 16)][0]
    endw = starts_v[pl.ds(SEGS_PER_W, 16)][0]
    base8 = (s0 // 8) * 8
    nld = (endw - base8 + 511) // 512

    def ld(j, c):
        pltpu.sync_copy(vcol_hbm.at[pl.ds(base8 + j * 512, 512)],
                        vbuf_v.at[pl.ds(j * 512, 512)])
        return c

    lax.fori_loop(0, nld, ld, 0)
    lane = lax.iota(jnp.int32, 16)

    def seg_body(s, carry):
        sg = w * SEGS_PER_W + s
        slot = lax.rem(s, SPR)
        stv = starts_v[pl.ds(s, 16)]
        st = stv[0]
        en = stv[1]
        zr = N_NODES + lax.rem(sg, NPAD - N_NODES)
        keys0 = jnp.full((16,), 3.0, jnp.float32)
        ids0 = jnp.broadcast_to(zr, (16,))

        def node_body(i, kki):
            kk, ii = kki
            v = -vbuf_v[pl.ds(i - base8, 16)][0]
            cond = jnp.logical_and(lane == 15, v < kk[15])
            nk = jnp.where(cond, v, kk)
            ni = jnp.where(cond, i, ii)
            snk, sni = lax.sort([nk, ni], dimension=0, num_keys=1)
            return (snk, sni)

        _, ids_fin = lax.fori_loop(st, en, node_body, (keys0, ids0))

        @pl.when(s >= SPR)
        def _():
            old = s - SPR
            pltpu.make_async_copy(rows_v.at[lax.rem(old, SPR)],
                                  out_hbm.at[pl.ds((w * SEGS_PER_W + old) * 16, 16)],
                                  sem_o).wait()

        @pl.when(s >= SPR - 1)
        def _():
            mid = s - (SPR - 1)
            mslot = lax.rem(mid, SPR)
            pltpu.make_async_copy(cs_hbm.at[ids_v.at[mslot]],
                                  rows_v.at[mslot], sem_g).wait()
            pltpu.async_copy(rows_v.at[mslot],
                             out_hbm.at[pl.ds((w * SEGS_PER_W + mid) * 16, 16)], sem_o)

        ids_v[slot] = ids_fin
        pltpu.async_copy(cs_hbm.at[ids_v.at[slot]], rows_v.at[slot], sem_g)
        return carry

    lax.fori_loop(0, SEGS_PER_W, seg_body, 0)

    def fin_gather(j, carry):
        jslot = lax.rem(j, SPR)
        pltpu.make_async_copy(cs_hbm.at[ids_v.at[jslot]], rows_v.at[jslot], sem_g).wait()
        pltpu.async_copy(rows_v.at[jslot],
                         out_hbm.at[pl.ds((w * SEGS_PER_W + j) * 16, 16)], sem_o)
        return carry

    lax.fori_loop(SEGS_PER_W - (SPR - 1), SEGS_PER_W, fin_gather, 0)

    def fin_store(j, carry):
        pltpu.make_async_copy(rows_v.at[lax.rem(j, SPR)],
                              out_hbm.at[pl.ds((w * SEGS_PER_W + j) * 16, 16)],
                              sem_o).wait()
        return carry

    lax.fori_loop(SEGS_PER_W - SPR, SEGS_PER_W, fin_store, 0)


@functools.lru_cache(maxsize=None)
def _sortpool_fn():
    return pl.kernel(
        _sortpool_body,
        out_type=jax.ShapeDtypeStruct((NSUB_PAD * 16, D2), jnp.float32),
        mesh=plsc.VectorSubcoreMesh(core_axis_name="c", subcore_axis_name="s"),
        compiler_params=pltpu.CompilerParams(use_tc_tiling_on_sc=False,
                                             needs_layout_passes=False),
        scratch_types=[
            pltpu.VMEM((64,), jnp.int32),
            pltpu.VMEM((VCOL_LEN,), jnp.float32),
            pltpu.VMEM((SPR, 16), jnp.int32),
            pltpu.VMEM((SPR, 16, D2), jnp.float32),
            pltpu.SemaphoreType.DMA,
            pltpu.SemaphoreType.DMA,
        ],
    )


def _sortpool(cs, vcol, starts):
    return _sortpool_fn()(cs, vcol, starts)


def _tc_head_body(d_ref, w1_ref, bc1_ref, w2_ref, bc2_ref, wl1_ref, bl1_ref,
                  wl2_ref, bl2_ref, s2g_ref, out_ref):
    z = jnp.dot(d_ref[...], w1_ref[...], preferred_element_type=jnp.float32)
    z = jax.nn.relu(z + bc1_ref[...][None, :])
    zm = z.reshape(NSUB_PAD * 8, 2, 16).max(axis=1)
    zm = zm.reshape(NSUB_PAD, 8, 16)
    zc = jnp.concatenate([zm[:, t:t + 4, :] for t in range(5)], axis=2)
    z2 = jnp.dot(zc.reshape(NSUB_PAD * 4, 80), w2_ref[...],
                 preferred_element_type=jnp.float32)
    z2 = jax.nn.relu(z2 + bc2_ref[...][None, :])
    z2v = z2.reshape(NSUB_PAD, 4, 32)
    g_iota = lax.broadcasted_iota(jnp.int32, (N_GRAPH, NSUB_PAD), 0)
    m = (s2g_ref[...][None, :] == g_iota).astype(jnp.float32)
    blocks = [jnp.dot(m, z2v[:, p, :], preferred_element_type=jnp.float32)
              for p in range(4)]
    sums = jnp.concatenate(blocks, axis=1)
    cnt = jnp.sum(m, axis=1, keepdims=True)
    g = sums / jnp.maximum(cnt, 1.0)
    g = jax.nn.relu(jnp.dot(g, wl1_ref[...], preferred_element_type=jnp.float32)
                    + bl1_ref[...][None, :])
    o = jnp.dot(g, wl2_ref[...], preferred_element_type=jnp.float32) + bl2_ref[...][None, :]
    mx = jnp.max(o, axis=-1, keepdims=True)
    lse = mx + jnp.log(jnp.sum(jnp.exp(o - mx), axis=-1, keepdims=True))
    out_ref[...] = o - lse


def _tc_head(dense, W1m, bc1, W2m, bc2, Wl1p, bl1, Wl2, bl2, s2g_pad):
    return pl.pallas_call(
        _tc_head_body,
        out_shape=jax.ShapeDtypeStruct((N_GRAPH, NUM_CLASSES), jnp.float32),
    )(dense, W1m, bc1, W2m, bc2, Wl1p, bl1, Wl2, bl2, s2g_pad)


def kernel(x, edge_index, node_to_subgraph, subgraph_to_graph,
           W0, b0, W1, b1, W2, b2, W3, b3,
           Wc1, bc1, Wc2, bc2, Wl1, bl1, Wl2, bl2):
    src, dst = edge_index[0], edge_index[1]
    pad_e = PAD_NODE + jnp.arange(E_PAD - N_EDGES, dtype=jnp.int32) % (NPAD - N_NODES)
    src_p = jnp.concatenate([src, pad_e]).reshape(NW * CPW, CHUNK)
    dst_p = jnp.concatenate([dst, pad_e]).reshape(NW * CPW, CHUNK)
    x_p = jnp.pad(x, ((0, NPAD - N_NODES), (0, 0)))

    n2s_pad = jnp.concatenate([node_to_subgraph,
                               jnp.full((NPAD - N_NODES,), 2000, jnp.int32)])
    v0, starts = _tc_xw0(x_p, W0, n2s_pad)
    deg_parts = _deg(dst_p)
    dinv, u = _tc_scale(deg_parts, v0)

    W3p = jnp.pad(W3, ((0, 0), (0, HIDDEN - 1)))
    b3p = jnp.pad(b3, (0, HIDDEN - 1))
    hs = []
    for b, Wn in ((b0, W1), (b1, W2), (b2, W3p)):
        parts = _msgpass(src_p, dst_p, u)
        h, u = _tc_layer(parts, u, dinv, b, Wn)
        hs.append(h)
    parts = _msgpass(src_p, dst_p, u)
    cs, vcol = _tc_cs(parts, u, dinv, b3p, hs[0], hs[1], hs[2])

    dense = _sortpool(cs, vcol, starts)

    W1m = jnp.pad(Wc1[:, 0, :].T, ((0, D2 - TOTAL_LATENT), (0, 0)))
    W2m = Wc2.transpose(2, 1, 0).reshape(80, 32)
    f = jnp.arange(128)
    perm = (f % 32) * 4 + f // 32
    Wl1p = Wl1[perm, :]
    s2g_pad = jnp.concatenate([subgraph_to_graph,
                               jnp.full((NSUB_PAD - N_SUB,), 1000, jnp.int32)])
    return _tc_head(dense, W1m, bc1, W2m, bc2, Wl1p, bl1, Wl2, bl2, s2g_pad)


# speedup vs baseline: 40.5030x; 1.0332x over previous
import functools

import jax
import jax.numpy as jnp
from jax import lax
from jax.experimental import pallas as pl
from jax.experimental.pallas import tpu as pltpu
from jax.experimental.pallas import tpu_sc as plsc

N_NODES = 10000
N_EDGES = 320000
N_SUB = 1000
N_GRAPH = 100
D_FEAT = 128
HIDDEN = 32
K = 16
TOTAL_LATENT = 97
NUM_CLASSES = 10

NPAD = 10240
NBUF = 12
PAD_NODE = N_NODES
NW = 32
CHUNK = 128
CPW = 80
E_PAD = NW * CPW * CHUNK
ROWS_PER_TILE = NPAD // 16


def _msgpass_body(src_hbm, dst_hbm, u_hbm, out_hbm, src_v, dst_v, rows_v, zbuf_v, acc_sh, sem_g, sem_s):
    cid = lax.axis_index("c")
    sid = lax.axis_index("s")
    wid = cid * 16 + sid
    pltpu.async_copy(src_hbm.at[pl.ds(wid * CPW, CPW)], src_v, sem_g)
    pltpu.async_copy(dst_hbm.at[pl.ds(wid * CPW, CPW)], dst_v, sem_g)

    def zf(i, c):
        zbuf_v[i // 2, pl.ds((i % 2) * 16, 16)] = jnp.zeros((16,), jnp.float32)
        return c

    lax.fori_loop(0, 2 * CHUNK, zf, 0)
    for j in range(ROWS_PER_TILE // CHUNK):
        pltpu.async_copy(zbuf_v, acc_sh.at[pl.ds(sid * ROWS_PER_TILE + j * CHUNK, CHUNK)], sem_s)
    pltpu.make_async_copy(src_hbm.at[pl.ds(wid * CPW, CPW)], src_v, sem_g).wait()
    pltpu.make_async_copy(dst_hbm.at[pl.ds(wid * CPW, CPW)], dst_v, sem_g).wait()
    for j in range(ROWS_PER_TILE // CHUNK):
        pltpu.make_async_copy(zbuf_v, acc_sh.at[pl.ds(sid * ROWS_PER_TILE + j * CHUNK, CHUNK)], sem_s).wait()
    plsc.subcore_barrier()

    for p in range(4):
        pltpu.async_copy(u_hbm.at[src_v.at[p]], rows_v.at[p], sem_g)

    def body(c, carry):
        slot = lax.rem(c, NBUF)
        pltpu.make_async_copy(u_hbm.at[src_v.at[c]], rows_v.at[slot], sem_g).wait()
        pltpu.async_copy(rows_v.at[slot], acc_sh.at[dst_v.at[c]], sem_s, add=True)

        @pl.when(c + 4 < CPW)
        def _():
            @pl.when(c >= NBUF - 4)
            def _():
                old = c - (NBUF - 4)
                pltpu.make_async_copy(rows_v.at[lax.rem(old, NBUF)],
                                      acc_sh.at[dst_v.at[old]], sem_s).wait()
            pltpu.async_copy(u_hbm.at[src_v.at[c + 4]],
                             rows_v.at[lax.rem(c + 4, NBUF)], sem_g)
        return carry

    lax.fori_loop(0, CPW, body, 0)

    def drain(j, carry):
        pltpu.make_async_copy(rows_v.at[lax.rem(j, NBUF)],
                              acc_sh.at[dst_v.at[j]], sem_s).wait()
        return carry

    lax.fori_loop(CPW - NBUF, CPW, drain, 0)
    plsc.subcore_barrier()
    pltpu.sync_copy(acc_sh.at[pl.ds(sid * ROWS_PER_TILE, ROWS_PER_TILE)],
                    out_hbm.at[cid, pl.ds(sid * ROWS_PER_TILE, ROWS_PER_TILE)])


@functools.lru_cache(maxsize=None)
def _msgpass_fn():
    return pl.kernel(
        _msgpass_body,
        out_type=jax.ShapeDtypeStruct((2, NPAD, HIDDEN), jnp.float32),
        mesh=plsc.VectorSubcoreMesh(core_axis_name="c", subcore_axis_name="s"),
        compiler_params=pltpu.CompilerParams(use_tc_tiling_on_sc=False),
        scratch_types=[
            pltpu.VMEM((CPW, CHUNK), jnp.int32),
            pltpu.VMEM((CPW, CHUNK), jnp.int32),
            pltpu.VMEM((NBUF, CHUNK, HIDDEN), jnp.float32),
            pltpu.VMEM((CHUNK, HIDDEN), jnp.float32),
            pltpu.VMEM_SHARED((NPAD, HIDDEN), jnp.float32),
            pltpu.SemaphoreType.DMA,
            pltpu.SemaphoreType.DMA,
        ],
    )


def _msgpass(src_p, dst_p, u):
    return _msgpass_fn()(src_p, dst_p, u)


def _deg_body(dst_hbm, out_hbm, dst_v, ones_v, zbuf_v, acc_sh):
    cid = lax.axis_index("c")
    sid = lax.axis_index("s")
    wid = cid * 16 + sid
    pltpu.sync_copy(dst_hbm.at[pl.ds(wid * CPW, CPW)], dst_v)

    def zf(i, c):
        zbuf_v[pl.ds(i * 16, 16)] = jnp.zeros((16,), jnp.float32)
        ones_v[pl.ds(i * 16, 16)] = jnp.ones((16,), jnp.float32)
        return c

    lax.fori_loop(0, CHUNK // 16, zf, 0)
    for j in range(ROWS_PER_TILE // CHUNK):
        pltpu.sync_copy(zbuf_v, acc_sh.at[pl.ds(sid * ROWS_PER_TILE + j * CHUNK, CHUNK)])
    plsc.subcore_barrier()

    def body(c, carry):
        pltpu.sync_copy(ones_v, acc_sh.at[dst_v.at[c]], add=True)
        return carry

    lax.fori_loop(0, CPW, body, 0)
    plsc.subcore_barrier()
    pltpu.sync_copy(acc_sh.at[pl.ds(sid * ROWS_PER_TILE, ROWS_PER_TILE)],
                    out_hbm.at[cid, pl.ds(sid * ROWS_PER_TILE, ROWS_PER_TILE)])


@functools.lru_cache(maxsize=None)
def _deg_fn():
    return pl.kernel(
        _deg_body,
        out_type=jax.ShapeDtypeStruct((2, NPAD), jnp.float32),
        mesh=plsc.VectorSubcoreMesh(core_axis_name="c", subcore_axis_name="s"),
        compiler_params=pltpu.CompilerParams(use_tc_tiling_on_sc=False),
        scratch_types=[
            pltpu.VMEM((CPW, CHUNK), jnp.int32),
            pltpu.VMEM((CHUNK,), jnp.float32),
            pltpu.VMEM((CHUNK,), jnp.float32),
            pltpu.VMEM_SHARED((NPAD,), jnp.float32),
        ],
    )


def _deg(dst_p):
    return _deg_fn()(dst_p)


NSTART = 1040


def _tc_xw0_body(x_ref, w0_ref, n2s_ref, v0_ref, starts_ref):
    v0_ref[...] = jnp.dot(x_ref[...], w0_ref[...], preferred_element_type=jnp.float32)
    s_iota = lax.broadcasted_iota(jnp.int32, (NSTART, 1), 0)
    acc = jnp.zeros((NSTART,), jnp.float32)
    for c in range(NPAD // 1024):
        chunk = n2s_ref[pl.ds(c * 1024, 1024)]
        acc = acc + jnp.sum((chunk[None, :] < s_iota).astype(jnp.float32), axis=1)
    starts_ref[...] = acc.astype(jnp.int32)


def _tc_xw0(x_p, W0, n2s_pad):
    return pl.pallas_call(
        _tc_xw0_body,
        out_shape=(jax.ShapeDtypeStruct((NPAD, HIDDEN), jnp.float32),
                   jax.ShapeDtypeStruct((NSTART,), jnp.int32)),
    )(x_p, W0, n2s_pad)


def _tc_scale_body(dp_ref, v0_ref, dinv_ref, u0_ref):
    dp = dp_ref[...]
    deg = 1.0 + (dp[0] + dp[1])[:, None]
    dinv = lax.rsqrt(deg)
    dinv_ref[...] = dinv
    u0_ref[...] = dinv * v0_ref[...]


def _tc_scale(deg_parts, v0):
    return pl.pallas_call(
        _tc_scale_body,
        out_shape=(jax.ShapeDtypeStruct((NPAD, 1), jnp.float32),
                   jax.ShapeDtypeStruct((NPAD, HIDDEN), jnp.float32)),
    )(deg_parts, v0)


def _tc_layer_body(dp_ref, u_ref, dinv_ref, b_ref, wn_ref, h_ref, un_ref):
    dp = dp_ref[...]
    dinv = dinv_ref[...]
    h = jnp.tanh(dinv * (dp[0] + dp[1] + u_ref[...]) + b_ref[...][None, :])
    h_ref[...] = h
    un_ref[...] = dinv * jnp.dot(h, wn_ref[...], preferred_element_type=jnp.float32)


def _tc_layer(parts, u, dinv, b, Wn):
    return pl.pallas_call(
        _tc_layer_body,
        out_shape=(jax.ShapeDtypeStruct((NPAD, HIDDEN), jnp.float32),
                   jax.ShapeDtypeStruct((NPAD, HIDDEN), jnp.float32)),
    )(parts, u, dinv, b, Wn)


D2 = 104
VCOL_LEN = NPAD + 512


def _tc_cs_body(dp_ref, u_ref, dinv_ref, b_ref, h0_ref, h1_ref, h2_ref,
                cs_ref, vcol_ref):
    dp = dp_ref[...]
    h3 = jnp.tanh(dinv_ref[...] * (dp[0] + dp[1] + u_ref[...]) + b_ref[...][None, :])
    row = lax.broadcasted_iota(jnp.int32, (NPAD, 1), 0)
    valid = (row < N_NODES).astype(jnp.float32)
    cs = jnp.concatenate(
        [h0_ref[...], h1_ref[...], h2_ref[...], h3[:, 0:1],
         jnp.zeros((NPAD, D2 - TOTAL_LATENT), jnp.float32)], axis=1)
    cs = cs * valid
    cs_ref[...] = cs
    vcol_ref[...] = jnp.concatenate([cs[:, TOTAL_LATENT - 1],
                                     jnp.zeros((VCOL_LEN - NPAD,), jnp.float32)])


def _tc_cs(parts, u, dinv, b, h0, h1, h2):
    return pl.pallas_call(
        _tc_cs_body,
        out_shape=(jax.ShapeDtypeStruct((NPAD, D2), jnp.float32),
                   jax.ShapeDtypeStruct((VCOL_LEN,), jnp.float32)),
    )(parts, u, dinv, b, h0, h1, h2)


SEGS_PER_W = 32
NSUB_PAD = 1024
SPR = 4


def _sortpool_body(cs_hbm, vcol_hbm, starts_hbm, out_hbm,
                   starts_v, vbuf_v, ids_v, rows_v, sem_g, sem_o):
    cid = lax.axis_index("c")
    sid = lax.axis_index("s")
    w = cid * 16 + sid
    pltpu.sync_copy(starts_hbm.at[pl.ds(w * SEGS_PER_W, 40)], starts_v.at[pl.ds(0, 40)])
    s0 = starts_v[pl.ds(0, 16)][0]
    endw = starts_v[pl.ds(SEGS_PER_W, 16)][0]
    base8 = (s0 // 8) * 8
    nld = (endw - base8 + 511) // 512

    def ld(j, c):
        pltpu.sync_copy(vcol_hbm.at[pl.ds(base8 + j * 512, 512)],
                        vbuf_v.at[pl.ds(j * 512, 512)])
        return c

    lax.fori_loop(0, nld, ld, 0)
    lane = lax.iota(jnp.int32, 16)

    def seg_body(s, carry):
        sg = w * SEGS_PER_W + s
        slot = lax.rem(s, SPR)
        stv = starts_v[pl.ds(s, 16)]
        st = stv[0]
        en = stv[1]
        zr = N_NODES + lax.rem(sg, NPAD - N_NODES)
        keys0 = jnp.full((16,), 3.0, jnp.float32)
        ids0 = jnp.broadcast_to(zr, (16,))

        def node_body(i, kki):
            kk, ii = kki
            v = -vbuf_v[pl.ds(i - base8, 16)][0]
            cond = jnp.logical_and(lane == 15, v < kk[15])
            nk = jnp.where(cond, v, kk)
            ni = jnp.where(cond, i, ii)
            snk, sni = lax.sort([nk, ni], dimension=0, num_keys=1)
            return (snk, sni)

        _, ids_fin = lax.fori_loop(st, en, node_body, (keys0, ids0))

        @pl.when(s >= SPR)
        def _():
            old = s - SPR
            pltpu.make_async_copy(rows_v.at[lax.rem(old, SPR)],
                                  out_hbm.at[pl.ds((w * SEGS_PER_W + old) * 16, 16)],
                                  sem_o).wait()

        @pl.when(s >= SPR - 1)
        def _():
            mid = s - (SPR - 1)
            mslot = lax.rem(mid, SPR)
            pltpu.make_async_copy(cs_hbm.at[ids_v.at[mslot]],
                                  rows_v.at[mslot], sem_g).wait()
            pltpu.async_copy(rows_v.at[mslot],
                             out_hbm.at[pl.ds((w * SEGS_PER_W + mid) * 16, 16)], sem_o)

        ids_v[slot] = ids_fin
        pltpu.async_copy(cs_hbm.at[ids_v.at[slot]], rows_v.at[slot], sem_g)
        return carry

    lax.fori_loop(0, SEGS_PER_W, seg_body, 0)

    def fin_gather(j, carry):
        jslot = lax.rem(j, SPR)
        pltpu.make_async_copy(cs_hbm.at[ids_v.at[jslot]], rows_v.at[jslot], sem_g).wait()
        pltpu.async_copy(rows_v.at[jslot],
                         out_hbm.at[pl.ds((w * SEGS_PER_W + j) * 16, 16)], sem_o)
        return carry

    lax.fori_loop(SEGS_PER_W - (SPR - 1), SEGS_PER_W, fin_gather, 0)

    def fin_store(j, carry):
        pltpu.make_async_copy(rows_v.at[lax.rem(j, SPR)],
                              out_hbm.at[pl.ds((w * SEGS_PER_W + j) * 16, 16)],
                              sem_o).wait()
        return carry

    lax.fori_loop(SEGS_PER_W - SPR, SEGS_PER_W, fin_store, 0)


@functools.lru_cache(maxsize=None)
def _sortpool_fn():
    return pl.kernel(
        _sortpool_body,
        out_type=jax.ShapeDtypeStruct((NSUB_PAD * 16, D2), jnp.float32),
        mesh=plsc.VectorSubcoreMesh(core_axis_name="c", subcore_axis_name="s"),
        compiler_params=pltpu.CompilerParams(use_tc_tiling_on_sc=False,
                                             needs_layout_passes=False),
        scratch_types=[
            pltpu.VMEM((64,), jnp.int32),
            pltpu.VMEM((VCOL_LEN,), jnp.float32),
            pltpu.VMEM((SPR, 16), jnp.int32),
            pltpu.VMEM((SPR, 16, D2), jnp.float32),
            pltpu.SemaphoreType.DMA,
            pltpu.SemaphoreType.DMA,
        ],
    )


def _sortpool(cs, vcol, starts):
    return _sortpool_fn()(cs, vcol, starts)


def _tc_head_body(d_ref, w1_ref, bc1_ref, w2_ref, bc2_ref, wl1_ref, bl1_ref,
                  wl2_ref, bl2_ref, s2g_ref, out_ref):
    z = jnp.dot(d_ref[...], w1_ref[...], preferred_element_type=jnp.float32)
    z = jax.nn.relu(z + bc1_ref[...][None, :])
    zm = z.reshape(NSUB_PAD * 8, 2, 16).max(axis=1)
    zm = zm.reshape(NSUB_PAD, 8, 16)
    zc = jnp.concatenate([zm[:, t:t + 4, :] for t in range(5)], axis=2)
    z2 = jnp.dot(zc.reshape(NSUB_PAD * 4, 80), w2_ref[...],
                 preferred_element_type=jnp.float32)
    z2 = jax.nn.relu(z2 + bc2_ref[...][None, :])
    z2v = z2.reshape(NSUB_PAD, 4, 32)
    g_iota = lax.broadcasted_iota(jnp.int32, (N_GRAPH, NSUB_PAD), 0)
    m = (s2g_ref[...][None, :] == g_iota).astype(jnp.float32)
    blocks = [jnp.dot(m, z2v[:, p, :], preferred_element_type=jnp.float32)
              for p in range(4)]
    sums = jnp.concatenate(blocks, axis=1)
    cnt = jnp.sum(m, axis=1, keepdims=True)
    g = sums / jnp.maximum(cnt, 1.0)
    g = jax.nn.relu(jnp.dot(g, wl1_ref[...], preferred_element_type=jnp.float32)
                    + bl1_ref[...][None, :])
    o = jnp.dot(g, wl2_ref[...], preferred_element_type=jnp.float32) + bl2_ref[...][None, :]
    mx = jnp.max(o, axis=-1, keepdims=True)
    lse = mx + jnp.log(jnp.sum(jnp.exp(o - mx), axis=-1, keepdims=True))
    out_ref[...] = o - lse


def _tc_head(dense, W1m, bc1, W2m, bc2, Wl1p, bl1, Wl2, bl2, s2g_pad):
    return pl.pallas_call(
        _tc_head_body,
        out_shape=jax.ShapeDtypeStruct((N_GRAPH, NUM_CLASSES), jnp.float32),
    )(dense, W1m, bc1, W2m, bc2, Wl1p, bl1, Wl2, bl2, s2g_pad)


def kernel(x, edge_index, node_to_subgraph, subgraph_to_graph,
           W0, b0, W1, b1, W2, b2, W3, b3,
           Wc1, bc1, Wc2, bc2, Wl1, bl1, Wl2, bl2):
    src, dst = edge_index[0], edge_index[1]
    pad_e = PAD_NODE + jnp.arange(E_PAD - N_EDGES, dtype=jnp.int32) % (NPAD - N_NODES)
    src_p = jnp.concatenate([src, pad_e]).reshape(NW * CPW, CHUNK)
    dst_p = jnp.concatenate([dst, pad_e]).reshape(NW * CPW, CHUNK)
    x_p = jnp.pad(x, ((0, NPAD - N_NODES), (0, 0)))

    n2s_pad = jnp.concatenate([node_to_subgraph,
                               jnp.full((NPAD - N_NODES,), 2000, jnp.int32)])
    v0, starts = _tc_xw0(x_p, W0, n2s_pad)
    deg_parts = _deg(dst_p)
    dinv, u = _tc_scale(deg_parts, v0)

    W3p = jnp.pad(W3, ((0, 0), (0, HIDDEN - 1)))
    b3p = jnp.pad(b3, (0, HIDDEN - 1))
    hs = []
    for b, Wn in ((b0, W1), (b1, W2), (b2, W3p)):
        parts = _msgpass(src_p, dst_p, u)
        h, u = _tc_layer(parts, u, dinv, b, Wn)
        hs.append(h)
    parts = _msgpass(src_p, dst_p, u)
    cs, vcol = _tc_cs(parts, u, dinv, b3p, hs[0], hs[1], hs[2])

    dense = _sortpool(cs, vcol, starts)

    W1m = jnp.pad(Wc1[:, 0, :].T, ((0, D2 - TOTAL_LATENT), (0, 0)))
    W2m = Wc2.transpose(2, 1, 0).reshape(80, 32)
    f = jnp.arange(128)
    perm = (f % 32) * 4 + f // 32
    Wl1p = Wl1[perm, :]
    s2g_pad = jnp.concatenate([subgraph_to_graph,
                               jnp.full((NSUB_PAD - N_SUB,), 1000, jnp.int32)])
    return _tc_head(dense, W1m, bc1, W2m, bc2, Wl1p, bl1, Wl2, bl2, s2g_pad)


# speedup vs baseline: 40.8269x; 1.0080x over previous
import functools

import jax
import jax.numpy as jnp
from jax import lax
from jax.experimental import pallas as pl
from jax.experimental.pallas import tpu as pltpu
from jax.experimental.pallas import tpu_sc as plsc

N_NODES = 10000
N_EDGES = 320000
N_SUB = 1000
N_GRAPH = 100
D_FEAT = 128
HIDDEN = 32
K = 16
TOTAL_LATENT = 97
NUM_CLASSES = 10

NPAD = 10240
NBUF = 16
PAD_NODE = N_NODES
NW = 32
CHUNK = 128
CPW = 80
E_PAD = NW * CPW * CHUNK
ROWS_PER_TILE = NPAD // 16


def _msgpass_body(src_hbm, dst_hbm, u_hbm, out_hbm, src_v, dst_v, rows_v, zbuf_v, acc_sh, sem_g, sem_s):
    cid = lax.axis_index("c")
    sid = lax.axis_index("s")
    wid = cid * 16 + sid
    pltpu.async_copy(src_hbm.at[pl.ds(wid * CPW, CPW)], src_v, sem_g)
    pltpu.async_copy(dst_hbm.at[pl.ds(wid * CPW, CPW)], dst_v, sem_g)

    def zf(i, c):
        zbuf_v[i // 2, pl.ds((i % 2) * 16, 16)] = jnp.zeros((16,), jnp.float32)
        return c

    lax.fori_loop(0, 2 * CHUNK, zf, 0)
    for j in range(ROWS_PER_TILE // CHUNK):
        pltpu.async_copy(zbuf_v, acc_sh.at[pl.ds(sid * ROWS_PER_TILE + j * CHUNK, CHUNK)], sem_s)
    pltpu.make_async_copy(src_hbm.at[pl.ds(wid * CPW, CPW)], src_v, sem_g).wait()
    pltpu.make_async_copy(dst_hbm.at[pl.ds(wid * CPW, CPW)], dst_v, sem_g).wait()
    for j in range(ROWS_PER_TILE // CHUNK):
        pltpu.make_async_copy(zbuf_v, acc_sh.at[pl.ds(sid * ROWS_PER_TILE + j * CHUNK, CHUNK)], sem_s).wait()
    plsc.subcore_barrier()

    for p in range(6):
        pltpu.async_copy(u_hbm.at[src_v.at[p]], rows_v.at[p], sem_g)

    def body(c, carry):
        slot = lax.rem(c, NBUF)
        pltpu.make_async_copy(u_hbm.at[src_v.at[c]], rows_v.at[slot], sem_g).wait()
        pltpu.async_copy(rows_v.at[slot], acc_sh.at[dst_v.at[c]], sem_s, add=True)

        @pl.when(c + 6 < CPW)
        def _():
            @pl.when(c >= NBUF - 6)
            def _():
                old = c - (NBUF - 6)
                pltpu.make_async_copy(rows_v.at[lax.rem(old, NBUF)],
                                      acc_sh.at[dst_v.at[old]], sem_s).wait()
            pltpu.async_copy(u_hbm.at[src_v.at[c + 6]],
                             rows_v.at[lax.rem(c + 6, NBUF)], sem_g)
        return carry

    lax.fori_loop(0, CPW, body, 0)

    def drain(j, carry):
        pltpu.make_async_copy(rows_v.at[lax.rem(j, NBUF)],
                              acc_sh.at[dst_v.at[j]], sem_s).wait()
        return carry

    lax.fori_loop(CPW - NBUF, CPW, drain, 0)
    plsc.subcore_barrier()
    pltpu.sync_copy(acc_sh.at[pl.ds(sid * ROWS_PER_TILE, ROWS_PER_TILE)],
                    out_hbm.at[cid, pl.ds(sid * ROWS_PER_TILE, ROWS_PER_TILE)])


@functools.lru_cache(maxsize=None)
def _msgpass_fn():
    return pl.kernel(
        _msgpass_body,
        out_type=jax.ShapeDtypeStruct((2, NPAD, HIDDEN), jnp.float32),
        mesh=plsc.VectorSubcoreMesh(core_axis_name="c", subcore_axis_name="s"),
        compiler_params=pltpu.CompilerParams(use_tc_tiling_on_sc=False),
        scratch_types=[
            pltpu.VMEM((CPW, CHUNK), jnp.int32),
            pltpu.VMEM((CPW, CHUNK), jnp.int32),
            pltpu.VMEM((NBUF, CHUNK, HIDDEN), jnp.float32),
            pltpu.VMEM((CHUNK, HIDDEN), jnp.float32),
            pltpu.VMEM_SHARED((NPAD, HIDDEN), jnp.float32),
            pltpu.SemaphoreType.DMA,
            pltpu.SemaphoreType.DMA,
        ],
    )


def _msgpass(src_p, dst_p, u):
    return _msgpass_fn()(src_p, dst_p, u)


def _deg_body(dst_hbm, out_hbm, dst_v, ones_v, zbuf_v, acc_sh):
    cid = lax.axis_index("c")
    sid = lax.axis_index("s")
    wid = cid * 16 + sid
    pltpu.sync_copy(dst_hbm.at[pl.ds(wid * CPW, CPW)], dst_v)

    def zf(i, c):
        zbuf_v[pl.ds(i * 16, 16)] = jnp.zeros((16,), jnp.float32)
        ones_v[pl.ds(i * 16, 16)] = jnp.ones((16,), jnp.float32)
        return c

    lax.fori_loop(0, CHUNK // 16, zf, 0)
    for j in range(ROWS_PER_TILE // CHUNK):
        pltpu.sync_copy(zbuf_v, acc_sh.at[pl.ds(sid * ROWS_PER_TILE + j * CHUNK, CHUNK)])
    plsc.subcore_barrier()

    def body(c, carry):
        pltpu.sync_copy(ones_v, acc_sh.at[dst_v.at[c]], add=True)
        return carry

    lax.fori_loop(0, CPW, body, 0)
    plsc.subcore_barrier()
    pltpu.sync_copy(acc_sh.at[pl.ds(sid * ROWS_PER_TILE, ROWS_PER_TILE)],
                    out_hbm.at[cid, pl.ds(sid * ROWS_PER_TILE, ROWS_PER_TILE)])


@functools.lru_cache(maxsize=None)
def _deg_fn():
    return pl.kernel(
        _deg_body,
        out_type=jax.ShapeDtypeStruct((2, NPAD), jnp.float32),
        mesh=plsc.VectorSubcoreMesh(core_axis_name="c", subcore_axis_name="s"),
        compiler_params=pltpu.CompilerParams(use_tc_tiling_on_sc=False),
        scratch_types=[
            pltpu.VMEM((CPW, CHUNK), jnp.int32),
            pltpu.VMEM((CHUNK,), jnp.float32),
            pltpu.VMEM((CHUNK,), jnp.float32),
            pltpu.VMEM_SHARED((NPAD,), jnp.float32),
        ],
    )


def _deg(dst_p):
    return _deg_fn()(dst_p)


NSTART = 1040


def _tc_xw0_body(x_ref, w0_ref, n2s_ref, v0_ref, starts_ref):
    v0_ref[...] = jnp.dot(x_ref[...], w0_ref[...], preferred_element_type=jnp.float32)
    s_iota = lax.broadcasted_iota(jnp.int32, (NSTART, 1), 0)
    acc = jnp.zeros((NSTART,), jnp.float32)
    for c in range(NPAD // 1024):
        chunk = n2s_ref[pl.ds(c * 1024, 1024)]
        acc = acc + jnp.sum((chunk[None, :] < s_iota).astype(jnp.float32), axis=1)
    starts_ref[...] = acc.astype(jnp.int32)


def _tc_xw0(x_p, W0, n2s_pad):
    return pl.pallas_call(
        _tc_xw0_body,
        out_shape=(jax.ShapeDtypeStruct((NPAD, HIDDEN), jnp.float32),
                   jax.ShapeDtypeStruct((NSTART,), jnp.int32)),
    )(x_p, W0, n2s_pad)


def _tc_scale_body(dp_ref, v0_ref, dinv_ref, u0_ref):
    dp = dp_ref[...]
    deg = 1.0 + (dp[0] + dp[1])[:, None]
    dinv = lax.rsqrt(deg)
    dinv_ref[...] = dinv
    u0_ref[...] = dinv * v0_ref[...]


def _tc_scale(deg_parts, v0):
    return pl.pallas_call(
        _tc_scale_body,
        out_shape=(jax.ShapeDtypeStruct((NPAD, 1), jnp.float32),
                   jax.ShapeDtypeStruct((NPAD, HIDDEN), jnp.float32)),
    )(deg_parts, v0)


def _tc_layer_body(dp_ref, u_ref, dinv_ref, b_ref, wn_ref, h_ref, un_ref):
    dp = dp_ref[...]
    dinv = dinv_ref[...]
    h = jnp.tanh(dinv * (dp[0] + dp[1] + u_ref[...]) + b_ref[...][None, :])
    h_ref[...] = h
    un_ref[...] = dinv * jnp.dot(h, wn_ref[...], preferred_element_type=jnp.float32)


def _tc_layer(parts, u, dinv, b, Wn):
    return pl.pallas_call(
        _tc_layer_body,
        out_shape=(jax.ShapeDtypeStruct((NPAD, HIDDEN), jnp.float32),
                   jax.ShapeDtypeStruct((NPAD, HIDDEN), jnp.float32)),
    )(parts, u, dinv, b, Wn)


D2 = 104
VCOL_LEN = NPAD + 512


def _tc_cs_body(dp_ref, u_ref, dinv_ref, b_ref, h0_ref, h1_ref, h2_ref,
                cs_ref, vcol_ref):
    dp = dp_ref[...]
    h3 = jnp.tanh(dinv_ref[...] * (dp[0] + dp[1] + u_ref[...]) + b_ref[...][None, :])
    row = lax.broadcasted_iota(jnp.int32, (NPAD, 1), 0)
    valid = (row < N_NODES).astype(jnp.float32)
    cs = jnp.concatenate(
        [h0_ref[...], h1_ref[...], h2_ref[...], h3[:, 0:1],
         jnp.zeros((NPAD, D2 - TOTAL_LATENT), jnp.float32)], axis=1)
    cs = cs * valid
    cs_ref[...] = cs
    vcol_ref[...] = jnp.concatenate([cs[:, TOTAL_LATENT - 1],
                                     jnp.zeros((VCOL_LEN - NPAD,), jnp.float32)])


def _tc_cs(parts, u, dinv, b, h0, h1, h2):
    return pl.pallas_call(
        _tc_cs_body,
        out_shape=(jax.ShapeDtypeStruct((NPAD, D2), jnp.float32),
                   jax.ShapeDtypeStruct((VCOL_LEN,), jnp.float32)),
    )(parts, u, dinv, b, h0, h1, h2)


SEGS_PER_W = 32
NSUB_PAD = 1024
SPR = 4


def _sortpool_body(cs_hbm, vcol_hbm, starts_hbm, out_hbm,
                   starts_v, vbuf_v, ids_v, rows_v, sem_g, sem_o):
    cid = lax.axis_index("c")
    sid = lax.axis_index("s")
    w = cid * 16 + sid
    pltpu.sync_copy(starts_hbm.at[pl.ds(w * SEGS_PER_W, 40)], starts_v.at[pl.ds(0, 40)])
    s0 = starts_v[pl.ds(0, 16)][0]
    endw = starts_v[pl.ds(SEGS_PER_W, 16)][0]
    base8 = (s0 // 8) * 8
    nld = (endw - base8 + 511) // 512

    def ld(j, c):
        pltpu.sync_copy(vcol_hbm.at[pl.ds(base8 + j * 512, 512)],
                        vbuf_v.at[pl.ds(j * 512, 512)])
        return c

    lax.fori_loop(0, nld, ld, 0)
    lane = lax.iota(jnp.int32, 16)

    def seg_body(s, carry):
        sg = w * SEGS_PER_W + s
        slot = lax.rem(s, SPR)
        stv = starts_v[pl.ds(s, 16)]
        st = stv[0]
        en = stv[1]
        zr = N_NODES + lax.rem(sg, NPAD - N_NODES)
        keys0 = jnp.full((16,), 3.0, jnp.float32)
        ids0 = jnp.broadcast_to(zr, (16,))

        def node_body(i, kki):
            kk, ii = kki
            v = -vbuf_v[pl.ds(i - base8, 16)][0]
            cond = jnp.logical_and(lane == 15, v < kk[15])
            nk = jnp.where(cond, v, kk)
            ni = jnp.where(cond, i, ii)
            snk, sni = lax.sort([nk, ni], dimension=0, num_keys=1)
            return (snk, sni)

        _, ids_fin = lax.fori_loop(st, en, node_body, (keys0, ids0))

        @pl.when(s >= SPR)
        def _():
            old = s - SPR
            pltpu.make_async_copy(rows_v.at[lax.rem(old, SPR)],
                                  out_hbm.at[pl.ds((w * SEGS_PER_W + old) * 16, 16)],
                                  sem_o).wait()

        @pl.when(s >= SPR - 1)
        def _():
            mid = s - (SPR - 1)
            mslot = lax.rem(mid, SPR)
            pltpu.make_async_copy(cs_hbm.at[ids_v.at[mslot]],
                                  rows_v.at[mslot], sem_g).wait()
            pltpu.async_copy(rows_v.at[mslot],
                             out_hbm.at[pl.ds((w * SEGS_PER_W + mid) * 16, 16)], sem_o)

        ids_v[slot] = ids_fin
        pltpu.async_copy(cs_hbm.at[ids_v.at[slot]], rows_v.at[slot], sem_g)
        return carry

    lax.fori_loop(0, SEGS_PER_W, seg_body, 0)

    def fin_gather(j, carry):
        jslot = lax.rem(j, SPR)
        pltpu.make_async_copy(cs_hbm.at[ids_v.at[jslot]], rows_v.at[jslot], sem_g).wait()
        pltpu.async_copy(rows_v.at[jslot],
                         out_hbm.at[pl.ds((w * SEGS_PER_W + j) * 16, 16)], sem_o)
        return carry

    lax.fori_loop(SEGS_PER_W - (SPR - 1), SEGS_PER_W, fin_gather, 0)

    def fin_store(j, carry):
        pltpu.make_async_copy(rows_v.at[lax.rem(j, SPR)],
                              out_hbm.at[pl.ds((w * SEGS_PER_W + j) * 16, 16)],
                              sem_o).wait()
        return carry

    lax.fori_loop(SEGS_PER_W - SPR, SEGS_PER_W, fin_store, 0)


@functools.lru_cache(maxsize=None)
def _sortpool_fn():
    return pl.kernel(
        _sortpool_body,
        out_type=jax.ShapeDtypeStruct((NSUB_PAD * 16, D2), jnp.float32),
        mesh=plsc.VectorSubcoreMesh(core_axis_name="c", subcore_axis_name="s"),
        compiler_params=pltpu.CompilerParams(use_tc_tiling_on_sc=False,
                                             needs_layout_passes=False),
        scratch_types=[
            pltpu.VMEM((64,), jnp.int32),
            pltpu.VMEM((VCOL_LEN,), jnp.float32),
            pltpu.VMEM((SPR, 16), jnp.int32),
            pltpu.VMEM((SPR, 16, D2), jnp.float32),
            pltpu.SemaphoreType.DMA,
            pltpu.SemaphoreType.DMA,
        ],
    )


def _sortpool(cs, vcol, starts):
    return _sortpool_fn()(cs, vcol, starts)


def _tc_head_body(d_ref, w1_ref, bc1_ref, w2_ref, bc2_ref, wl1_ref, bl1_ref,
                  wl2_ref, bl2_ref, s2g_ref, out_ref):
    z = jnp.dot(d_ref[...], w1_ref[...], preferred_element_type=jnp.float32)
    z = jax.nn.relu(z + bc1_ref[...][None, :])
    zm = z.reshape(NSUB_PAD * 8, 2, 16).max(axis=1)
    zm = zm.reshape(NSUB_PAD, 8, 16)
    zc = jnp.concatenate([zm[:, t:t + 4, :] for t in range(5)], axis=2)
    z2 = jnp.dot(zc.reshape(NSUB_PAD * 4, 80), w2_ref[...],
                 preferred_element_type=jnp.float32)
    z2 = jax.nn.relu(z2 + bc2_ref[...][None, :])
    z2v = z2.reshape(NSUB_PAD, 4, 32)
    g_iota = lax.broadcasted_iota(jnp.int32, (N_GRAPH, NSUB_PAD), 0)
    m = (s2g_ref[...][None, :] == g_iota).astype(jnp.float32)
    blocks = [jnp.dot(m, z2v[:, p, :], preferred_element_type=jnp.float32)
              for p in range(4)]
    sums = jnp.concatenate(blocks, axis=1)
    cnt = jnp.sum(m, axis=1, keepdims=True)
    g = sums / jnp.maximum(cnt, 1.0)
    g = jax.nn.relu(jnp.dot(g, wl1_ref[...], preferred_element_type=jnp.float32)
                    + bl1_ref[...][None, :])
    o = jnp.dot(g, wl2_ref[...], preferred_element_type=jnp.float32) + bl2_ref[...][None, :]
    mx = jnp.max(o, axis=-1, keepdims=True)
    lse = mx + jnp.log(jnp.sum(jnp.exp(o - mx), axis=-1, keepdims=True))
    out_ref[...] = o - lse


def _tc_head(dense, W1m, bc1, W2m, bc2, Wl1p, bl1, Wl2, bl2, s2g_pad):
    return pl.pallas_call(
        _tc_head_body,
        out_shape=jax.ShapeDtypeStruct((N_GRAPH, NUM_CLASSES), jnp.float32),
    )(dense, W1m, bc1, W2m, bc2, Wl1p, bl1, Wl2, bl2, s2g_pad)


def kernel(x, edge_index, node_to_subgraph, subgraph_to_graph,
           W0, b0, W1, b1, W2, b2, W3, b3,
           Wc1, bc1, Wc2, bc2, Wl1, bl1, Wl2, bl2):
    src, dst = edge_index[0], edge_index[1]
    pad_e = PAD_NODE + jnp.arange(E_PAD - N_EDGES, dtype=jnp.int32) % (NPAD - N_NODES)
    src_p = jnp.concatenate([src, pad_e]).reshape(NW * CPW, CHUNK)
    dst_p = jnp.concatenate([dst, pad_e]).reshape(NW * CPW, CHUNK)
    x_p = jnp.pad(x, ((0, NPAD - N_NODES), (0, 0)))

    n2s_pad = jnp.concatenate([node_to_subgraph,
                               jnp.full((NPAD - N_NODES,), 2000, jnp.int32)])
    v0, starts = _tc_xw0(x_p, W0, n2s_pad)
    deg_parts = _deg(dst_p)
    dinv, u = _tc_scale(deg_parts, v0)

    W3p = jnp.pad(W3, ((0, 0), (0, HIDDEN - 1)))
    b3p = jnp.pad(b3, (0, HIDDEN - 1))
    hs = []
    for b, Wn in ((b0, W1), (b1, W2), (b2, W3p)):
        parts = _msgpass(src_p, dst_p, u)
        h, u = _tc_layer(parts, u, dinv, b, Wn)
        hs.append(h)
    parts = _msgpass(src_p, dst_p, u)
    cs, vcol = _tc_cs(parts, u, dinv, b3p, hs[0], hs[1], hs[2])

    dense = _sortpool(cs, vcol, starts)

    W1m = jnp.pad(Wc1[:, 0, :].T, ((0, D2 - TOTAL_LATENT), (0, 0)))
    W2m = Wc2.transpose(2, 1, 0).reshape(80, 32)
    f = jnp.arange(128)
    perm = (f % 32) * 4 + f // 32
    Wl1p = Wl1[perm, :]
    s2g_pad = jnp.concatenate([subgraph_to_graph,
                               jnp.full((NSUB_PAD - N_SUB,), 1000, jnp.int32)])
    return _tc_head(dense, W1m, bc1, W2m, bc2, Wl1p, bl1, Wl2, bl2, s2g_pad)
